# Initial kernel scaffold; baseline (speedup 1.0000x reference)
import jax
import jax.numpy as jnp
from jax.experimental import pallas as pl


def kernel(x, edge_index, batch, params):
    raise NotImplementedError("write your pallas kernel here")


# speedup vs baseline: 6.7664x; 6.7664x over previous
import functools

import jax
import jax.numpy as jnp
import numpy as np
from jax.experimental import pallas as pl
from jax.experimental.pallas import tpu as pltpu

N = 50000
E = 800000
HIDDEN = 64
HEADS = 8
HEAD_DIM = 8
NUM_GRAPHS = 64
NUM_BLOCKS = 3

EBLK = 8000
NBLK = 5000


def _bn_fold(bn):
    s = bn["g"] / jnp.sqrt(bn["rv"] + 1e-5)
    t = bn["b"] - bn["rm"] * s
    return s.reshape(1, -1), t.reshape(1, -1)


def _ln_expr(z, g, b):
    m = jnp.mean(z, axis=-1, keepdims=True)
    v = jnp.mean((z - m) ** 2, axis=-1, keepdims=True)
    return (z - m) * jax.lax.rsqrt(v + 1e-5) * g + b


def _enc_body(x_ref, w_ref, s_ref, t_ref, o_ref):
    y = jnp.dot(x_ref[...], w_ref[...], preferred_element_type=jnp.float32, precision=jax.lax.Precision.HIGHEST)
    o_ref[...] = jnp.maximum(y * s_ref[...] + t_ref[...], 0.0)


def _encoder(x, wT, s, t):
    return pl.pallas_call(
        _enc_body,
        grid=(N // NBLK,),
        in_specs=[
            pl.BlockSpec((NBLK, 4), lambda i: (i, 0)),
            pl.BlockSpec((4, HIDDEN), lambda i: (0, 0)),
            pl.BlockSpec((1, HIDDEN), lambda i: (0, 0)),
            pl.BlockSpec((1, HIDDEN), lambda i: (0, 0)),
        ],
        out_specs=pl.BlockSpec((NBLK, HIDDEN), lambda i: (i, 0)),
        out_shape=jax.ShapeDtypeStruct((N, HIDDEN), jnp.float32),
    )(x, wT, s, t)


def _edge_mlp_body(xr_ref, xc_ref, w1a_ref, w1b_ref, s1_ref, t1_ref,
                   w2_ref, s2_ref, t2_ref, o_ref):
    acc = jnp.dot(xr_ref[...], w1a_ref[...], preferred_element_type=jnp.float32, precision=jax.lax.Precision.HIGHEST)
    acc = acc + jnp.dot(xc_ref[...], w1b_ref[...], preferred_element_type=jnp.float32, precision=jax.lax.Precision.HIGHEST)
    h = jnp.maximum(acc * s1_ref[...] + t1_ref[...], 0.0)
    h2 = jnp.dot(h, w2_ref[...], preferred_element_type=jnp.float32, precision=jax.lax.Precision.HIGHEST)
    o_ref[...] = jnp.maximum(h2 * s2_ref[...] + t2_ref[...], 0.0)


def _edge_mlp(xr, xc, w1aT, w1bT, s1, t1, w2T, s2, t2):
    full = lambda i: (0, 0)
    return pl.pallas_call(
        _edge_mlp_body,
        grid=(E // EBLK,),
        in_specs=[
            pl.BlockSpec((EBLK, HIDDEN), lambda i: (i, 0)),
            pl.BlockSpec((EBLK, HIDDEN), lambda i: (i, 0)),
            pl.BlockSpec((HIDDEN, HIDDEN), full),
            pl.BlockSpec((HIDDEN, HIDDEN), full),
            pl.BlockSpec((1, HIDDEN), full),
            pl.BlockSpec((1, HIDDEN), full),
            pl.BlockSpec((HIDDEN, HIDDEN), full),
            pl.BlockSpec((1, HIDDEN), full),
            pl.BlockSpec((1, HIDDEN), full),
        ],
        out_specs=pl.BlockSpec((EBLK, HIDDEN), lambda i: (i, 0)),
        out_shape=jax.ShapeDtypeStruct((E, HIDDEN), jnp.float32),
    )(xr, xc, w1aT, w1bT, s1, t1, w2T, s2, t2)


def _ln1_body(y_ref, r_ref, g_ref, b_ref, o_ref):
    z = y_ref[...] + r_ref[...]
    o_ref[...] = _ln_expr(z, g_ref[...], b_ref[...])


def _res_ln(y, r, g, b):
    full = lambda i: (0, 0)
    return pl.pallas_call(
        _ln1_body,
        grid=(N // NBLK,),
        in_specs=[
            pl.BlockSpec((NBLK, HIDDEN), lambda i: (i, 0)),
            pl.BlockSpec((NBLK, HIDDEN), lambda i: (i, 0)),
            pl.BlockSpec((1, HIDDEN), full),
            pl.BlockSpec((1, HIDDEN), full),
        ],
        out_specs=pl.BlockSpec((NBLK, HIDDEN), lambda i: (i, 0)),
        out_shape=jax.ShapeDtypeStruct((N, HIDDEN), jnp.float32),
    )(y, r, g, b)


def _gatproj_body(x_ref, w_ref, ms_ref, md_ref, h_ref, as_ref, ad_ref,
                  cs_ref, cd_ref):
    i = pl.program_id(0)
    h = jnp.dot(x_ref[...], w_ref[...], preferred_element_type=jnp.float32, precision=jax.lax.Precision.HIGHEST)
    h_ref[...] = h
    a_s = jnp.dot(h, ms_ref[...], preferred_element_type=jnp.float32, precision=jax.lax.Precision.HIGHEST)
    a_d = jnp.dot(h, md_ref[...], preferred_element_type=jnp.float32, precision=jax.lax.Precision.HIGHEST)
    as_ref[...] = a_s
    ad_ref[...] = a_d
    bs = jnp.max(a_s, axis=0, keepdims=True)
    bd = jnp.max(a_d, axis=0, keepdims=True)

    @pl.when(i == 0)
    def _():
        cs_ref[...] = bs
        cd_ref[...] = bd

    @pl.when(i > 0)
    def _():
        cs_ref[...] = jnp.maximum(cs_ref[...], bs)
        cd_ref[...] = jnp.maximum(cd_ref[...], bd)


def _gat_proj(x, wT, msT, mdT):
    full = lambda i: (0, 0)
    return pl.pallas_call(
        _gatproj_body,
        grid=(N // NBLK,),
        in_specs=[
            pl.BlockSpec((NBLK, HIDDEN), lambda i: (i, 0)),
            pl.BlockSpec((HIDDEN, HIDDEN), full),
            pl.BlockSpec((HIDDEN, HEADS), full),
            pl.BlockSpec((HIDDEN, HEADS), full),
        ],
        out_specs=[
            pl.BlockSpec((NBLK, HIDDEN), lambda i: (i, 0)),
            pl.BlockSpec((NBLK, HEADS), lambda i: (i, 0)),
            pl.BlockSpec((NBLK, HEADS), lambda i: (i, 0)),
            pl.BlockSpec((1, HEADS), full),
            pl.BlockSpec((1, HEADS), full),
        ],
        out_shape=[
            jax.ShapeDtypeStruct((N, HIDDEN), jnp.float32),
            jax.ShapeDtypeStruct((N, HEADS), jnp.float32),
            jax.ShapeDtypeStruct((N, HEADS), jnp.float32),
            jax.ShapeDtypeStruct((1, HEADS), jnp.float32),
            jax.ShapeDtypeStruct((1, HEADS), jnp.float32),
        ],
    )(x, wT, msT, mdT)


def _attn_body(ae_ref, ad_ref, c_ref, o_ref):
    z = ae_ref[...] + ad_ref[...]
    a = jnp.where(z >= 0.0, z, 0.2 * z)
    o_ref[...] = jnp.exp(a - c_ref[...])


def _attn_edge(ae, ad, c):
    full = lambda i: (0, 0)
    return pl.pallas_call(
        _attn_body,
        grid=(E // EBLK,),
        in_specs=[
            pl.BlockSpec((EBLK, HEADS), lambda i: (i, 0)),
            pl.BlockSpec((EBLK, HEADS), lambda i: (i, 0)),
            pl.BlockSpec((1, HEADS), full),
        ],
        out_specs=pl.BlockSpec((EBLK, HEADS), lambda i: (i, 0)),
        out_shape=jax.ShapeDtypeStruct((E, HEADS), jnp.float32),
    )(ae, ad, c)


def _msg_body(h_ref, ex_ref, dd_ref, r_ref, o_ref):
    coef = ex_ref[...] / (dd_ref[...] + 1e-16)
    c64 = jnp.dot(coef, r_ref[...], preferred_element_type=jnp.float32, precision=jax.lax.Precision.HIGHEST)
    o_ref[...] = h_ref[...] * c64


def _msg_edge(h_src, ex, dd, rexp):
    full = lambda i: (0, 0)
    return pl.pallas_call(
        _msg_body,
        grid=(E // EBLK,),
        in_specs=[
            pl.BlockSpec((EBLK, HIDDEN), lambda i: (i, 0)),
            pl.BlockSpec((EBLK, HEADS), lambda i: (i, 0)),
            pl.BlockSpec((EBLK, HEADS), lambda i: (i, 0)),
            pl.BlockSpec((HEADS, HIDDEN), full),
        ],
        out_specs=pl.BlockSpec((EBLK, HIDDEN), lambda i: (i, 0)),
        out_shape=jax.ShapeDtypeStruct((E, HIDDEN), jnp.float32),
    )(h_src, ex, dd, rexp)


def _ln2_body(xg_ref, xp_ref, bias_ref, g1_ref, b1_ref, g2_ref, b2_ref, o_ref):
    t = _ln_expr(xg_ref[...] + bias_ref[...] + xp_ref[...], g1_ref[...], b1_ref[...])
    o_ref[...] = _ln_expr(t + xp_ref[...], g2_ref[...], b2_ref[...])


def _gat_epilogue(xg, xp, bias, g1, b1, g2, b2):
    full = lambda i: (0, 0)
    return pl.pallas_call(
        _ln2_body,
        grid=(N // NBLK,),
        in_specs=[
            pl.BlockSpec((NBLK, HIDDEN), lambda i: (i, 0)),
            pl.BlockSpec((NBLK, HIDDEN), lambda i: (i, 0)),
            pl.BlockSpec((1, HIDDEN), full),
            pl.BlockSpec((1, HIDDEN), full),
            pl.BlockSpec((1, HIDDEN), full),
            pl.BlockSpec((1, HIDDEN), full),
            pl.BlockSpec((1, HIDDEN), full),
        ],
        out_specs=pl.BlockSpec((NBLK, HIDDEN), lambda i: (i, 0)),
        out_shape=jax.ShapeDtypeStruct((N, HIDDEN), jnp.float32),
    )(xg, xp, bias, g1, b1, g2, b2)


def _pool_body(x_ref, b_ref, sum_ref, cnt_ref, max_ref):
    i = pl.program_id(0)
    x = x_ref[...]
    bb = b_ref[...]
    gids = jax.lax.broadcasted_iota(jnp.int32, (1, NUM_GRAPHS), 1)
    onehot = (bb == gids).astype(jnp.float32)
    psum = jax.lax.dot_general(onehot, x, (((0,), (0,)), ((), ())),
                               preferred_element_type=jnp.float32, precision=jax.lax.Precision.HIGHEST)
    ones = jnp.ones((x.shape[0], 1), jnp.float32)
    pcnt = jax.lax.dot_general(onehot, ones, (((0,), (0,)), ((), ())),
                               preferred_element_type=jnp.float32, precision=jax.lax.Precision.HIGHEST)
    neg = jnp.float32(-3.0e38)
    rows = []
    for g in range(NUM_GRAPHS):
        mask = bb == g
        rows.append(jnp.max(jnp.where(mask, x, neg), axis=0, keepdims=True))
    pmax = jnp.concatenate(rows, axis=0)

    @pl.when(i == 0)
    def _():
        sum_ref[...] = psum
        cnt_ref[...] = pcnt
        max_ref[...] = pmax

    @pl.when(i > 0)
    def _():
        sum_ref[...] = sum_ref[...] + psum
        cnt_ref[...] = cnt_ref[...] + pcnt
        max_ref[...] = jnp.maximum(max_ref[...], pmax)


def _pool(x, batch2d):
    full = lambda i: (0, 0)
    return pl.pallas_call(
        _pool_body,
        grid=(N // NBLK,),
        in_specs=[
            pl.BlockSpec((NBLK, HIDDEN), lambda i: (i, 0)),
            pl.BlockSpec((NBLK, 1), lambda i: (i, 0)),
        ],
        out_specs=[
            pl.BlockSpec((NUM_GRAPHS, HIDDEN), full),
            pl.BlockSpec((NUM_GRAPHS, 1), full),
            pl.BlockSpec((NUM_GRAPHS, HIDDEN), full),
        ],
        out_shape=[
            jax.ShapeDtypeStruct((NUM_GRAPHS, HIDDEN), jnp.float32),
            jax.ShapeDtypeStruct((NUM_GRAPHS, 1), jnp.float32),
            jax.ShapeDtypeStruct((NUM_GRAPHS, HIDDEN), jnp.float32),
        ],
    )(x, batch2d)


def _head_body(xsum_ref, cnt_ref, xmax_ref, wpa_ref, wpb_ref, bp_ref,
               wc1_ref, bc1_ref, wc2_ref, bc2_ref,
               we1_ref, be1_ref, we2_ref, be2_ref,
               logits_ref, energy_ref):
    cnt = jnp.maximum(cnt_ref[...], 1.0)
    xmean = xsum_ref[...] / cnt
    xm = xmax_ref[...]
    xm = jnp.where(xm > jnp.float32(-1.0e38), xm, 0.0)
    pool = jnp.dot(xmean, wpa_ref[...], preferred_element_type=jnp.float32, precision=jax.lax.Precision.HIGHEST)
    pool = pool + jnp.dot(xm, wpb_ref[...], preferred_element_type=jnp.float32, precision=jax.lax.Precision.HIGHEST)
    pool = jnp.maximum(pool + bp_ref[...], 0.0)
    h1 = jnp.maximum(
        jnp.dot(pool, wc1_ref[...], preferred_element_type=jnp.float32, precision=jax.lax.Precision.HIGHEST) + bc1_ref[...], 0.0)
    logits_ref[...] = jnp.dot(h1, wc2_ref[...], preferred_element_type=jnp.float32, precision=jax.lax.Precision.HIGHEST) + bc2_ref[...]
    e1 = jnp.maximum(
        jnp.dot(pool, we1_ref[...], preferred_element_type=jnp.float32, precision=jax.lax.Precision.HIGHEST) + be1_ref[...], 0.0)
    z = jnp.dot(e1, we2_ref[...], preferred_element_type=jnp.float32, precision=jax.lax.Precision.HIGHEST) + be2_ref[...]
    energy_ref[...] = jnp.maximum(z, 0.0) + jnp.log(1.0 + jnp.exp(-jnp.abs(z)))


def _heads(xsum, cnt, xmax, p):
    wpa = p["pool"]["W"][:, :HIDDEN].T
    wpb = p["pool"]["W"][:, HIDDEN:].T
    bp = p["pool"]["b"].reshape(1, -1)
    wc1 = p["cls1"]["W"].T
    bc1 = p["cls1"]["b"].reshape(1, -1)
    wc2 = p["cls2"]["W"].T
    bc2 = p["cls2"]["b"].reshape(1, -1)
    we1 = p["en1"]["W"].T
    be1 = p["en1"]["b"].reshape(1, -1)
    we2 = p["en2"]["W"].T
    be2 = p["en2"]["b"].reshape(1, -1)
    return pl.pallas_call(
        _head_body,
        out_shape=[
            jax.ShapeDtypeStruct((NUM_GRAPHS, 5), jnp.float32),
            jax.ShapeDtypeStruct((NUM_GRAPHS, 1), jnp.float32),
        ],
    )(xsum, cnt, xmax, wpa, wpb, bp, wc1, bc1, wc2, bc2, we1, be1, we2, be2)


def kernel(x, edge_index, batch, params):
    row = edge_index[0]
    col = edge_index[1]

    encw = params["enc_lin"]["W"].T
    s0, t0 = _bn_fold(params["enc_bn"])
    t0 = t0 + params["enc_lin"]["b"].reshape(1, -1) * s0
    h = _encoder(x, encw, s0, t0)

    for i in range(NUM_BLOCKS):
        pc = params["edge_convs"][i]
        w1 = pc["lin1"]["W"]
        w1aT = w1[:, :HIDDEN].T
        w1bT = w1[:, HIDDEN:].T
        s1, t1 = _bn_fold(pc["bn1"])
        t1 = t1 + pc["lin1"]["b"].reshape(1, -1) * s1
        w2T = pc["lin2"]["W"].T
        s2, t2 = _bn_fold(pc["bn2"])
        t2 = t2 + pc["lin2"]["b"].reshape(1, -1) * s2

        xr = jnp.take(h, row, axis=0)
        xc = jnp.take(h, col, axis=0)
        he = _edge_mlp(xr, xc, w1aT, w1bT, s1, t1, w2T, s2, t2)
        ec = jax.ops.segment_sum(he, row, num_segments=N)
        ln = params["lns"][2 * i]
        x1 = _res_ln(ec, h, ln["g"].reshape(1, -1), ln["b"].reshape(1, -1))

        g = params["gats"][i]
        msT = np.zeros((HIDDEN, HEADS), np.float32)
        ms = jnp.zeros((HEADS, HIDDEN), jnp.float32)
        idx_h = jnp.repeat(jnp.arange(HEADS), HEAD_DIM)
        ms = ms.at[idx_h, jnp.arange(HIDDEN)].set(g["att_src"].reshape(-1))
        md = jnp.zeros((HEADS, HIDDEN), jnp.float32)
        md = md.at[idx_h, jnp.arange(HIDDEN)].set(g["att_dst"].reshape(-1))
        hh, a_s, a_d, cs, cd = _gat_proj(x1, g["W"].T, ms.T, md.T)
        c = cs + cd
        ae = jnp.take(a_s, row, axis=0)
        ad = jnp.take(a_d, col, axis=0)
        ex = _attn_edge(ae, ad, c)
        denom = jax.ops.segment_sum(ex, col, num_segments=N)
        dd = jnp.take(denom, col, axis=0)
        h_src = jnp.take(hh, row, axis=0)
        rexp = jnp.zeros((HEADS, HIDDEN), jnp.float32)
        rexp = rexp.at[idx_h, jnp.arange(HIDDEN)].set(1.0)
        msg = _msg_edge(h_src, ex, dd, rexp)
        xg = jax.ops.segment_sum(msg, col, num_segments=N)
        ln2 = params["lns"][2 * i + 1]
        h = _gat_epilogue(xg, x1, g["bias"].reshape(1, -1),
                          g["ln_g"].reshape(1, -1), g["ln_b"].reshape(1, -1),
                          ln2["g"].reshape(1, -1), ln2["b"].reshape(1, -1))

    xsum, cnt, xmax = _pool(h, batch.reshape(-1, 1).astype(jnp.int32))
    logits, energy = _heads(xsum, cnt, xmax, params)
    return logits, energy


# speedup vs baseline: 12.6620x; 1.8713x over previous
import functools

import jax
import jax.numpy as jnp
import numpy as np
from jax import lax
from jax.experimental import pallas as pl
from jax.experimental.pallas import tpu as pltpu
from jax.experimental.pallas import tpu_sc as plsc

N = 50000
E = 800000
HIDDEN = 64
HEADS = 8
HEAD_DIM = 8
NUM_GRAPHS = 64
NUM_BLOCKS = 3

EBLK = 8000
NBLK = 5000


def _bn_fold(bn):
    s = bn["g"] / jnp.sqrt(bn["rv"] + 1e-5)
    t = bn["b"] - bn["rm"] * s
    return s.reshape(1, -1), t.reshape(1, -1)


def _ln_expr(z, g, b):
    m = jnp.mean(z, axis=-1, keepdims=True)
    v = jnp.mean((z - m) ** 2, axis=-1, keepdims=True)
    return (z - m) * jax.lax.rsqrt(v + 1e-5) * g + b


GCHUNK = 128
NCHUNKS = E // GCHUNK
NWORKERS = 32
GPAIRS = (NCHUNKS + 2 * NWORKERS - 1) // (2 * NWORKERS)


def _sc_gather_multi(pairs):
    dims = tuple(int(t.shape[1]) for t, _ in pairs)
    np_ = len(pairs)

    scratch = []
    for d in dims:
        scratch += [
            pltpu.VMEM((GCHUNK,), jnp.int32), pltpu.VMEM((GCHUNK,), jnp.int32),
            pltpu.VMEM((GCHUNK, d), jnp.float32), pltpu.VMEM((GCHUNK, d), jnp.float32),
            pltpu.SemaphoreType.DMA, pltpu.SemaphoreType.DMA,
            pltpu.SemaphoreType.DMA, pltpu.SemaphoreType.DMA,
        ]

    @functools.partial(
        pl.kernel,
        mesh=plsc.VectorSubcoreMesh(core_axis_name="c", subcore_axis_name="s"),
        out_type=[jax.ShapeDtypeStruct((E, d), jnp.float32) for d in dims],
        scratch_types=scratch,
        compiler_params=pltpu.CompilerParams(use_tc_tiling_on_sc=False),
    )
    def k(*refs):
        tables = refs[:np_]
        idxs = refs[np_:2 * np_]
        outs = refs[2 * np_:3 * np_]
        scr = refs[3 * np_:]
        w = lax.axis_index("s") * 2 + lax.axis_index("c")

        def pair_iter(jj, carry):
            for p in range(np_):
                ib = scr[8 * p:8 * p + 2]
                rb = scr[8 * p + 2:8 * p + 4]
                gs = scr[8 * p + 4:8 * p + 6]
                ws = scr[8 * p + 6:8 * p + 8]
                tab, idx, out = tables[p], idxs[p], outs[p]
                for b in range(2):
                    g = w + NWORKERS * (2 * jj + b)

                    @pl.when(g < NCHUNKS)
                    def _():
                        pltpu.sync_copy(idx.at[pl.ds(g * GCHUNK, GCHUNK)], ib[b])
                        pltpu.async_copy(tab.at[ib[b]], rb[b], gs[b])
                for b in range(2):
                    g = w + NWORKERS * (2 * jj + b)

                    @pl.when(g < NCHUNKS)
                    def _():
                        pltpu.make_async_copy(tab.at[ib[b]], rb[b], gs[b]).wait()
                        pltpu.async_copy(rb[b], out.at[pl.ds(g * GCHUNK, GCHUNK)], ws[b])
                for b in range(2):
                    g = w + NWORKERS * (2 * jj + b)

                    @pl.when(g < NCHUNKS)
                    def _():
                        pltpu.make_async_copy(
                            rb[b], out.at[pl.ds(g * GCHUNK, GCHUNK)], ws[b]).wait()
            return carry

        lax.fori_loop(0, GPAIRS, pair_iter, 0)

    res = k(*[t for t, _ in pairs], *[i for _, i in pairs])
    return list(res) if np_ > 1 else [res]


def _enc_body(x_ref, w_ref, s_ref, t_ref, o_ref):
    y = jnp.dot(x_ref[...], w_ref[...], preferred_element_type=jnp.float32, precision=jax.lax.Precision.HIGHEST)
    o_ref[...] = jnp.maximum(y * s_ref[...] + t_ref[...], 0.0)


def _encoder(x, wT, s, t):
    return pl.pallas_call(
        _enc_body,
        grid=(N // NBLK,),
        in_specs=[
            pl.BlockSpec((NBLK, 4), lambda i: (i, 0)),
            pl.BlockSpec((4, HIDDEN), lambda i: (0, 0)),
            pl.BlockSpec((1, HIDDEN), lambda i: (0, 0)),
            pl.BlockSpec((1, HIDDEN), lambda i: (0, 0)),
        ],
        out_specs=pl.BlockSpec((NBLK, HIDDEN), lambda i: (i, 0)),
        out_shape=jax.ShapeDtypeStruct((N, HIDDEN), jnp.float32),
    )(x, wT, s, t)


def _edge_mlp_body(xr_ref, xc_ref, w1a_ref, w1b_ref, s1_ref, t1_ref,
                   w2_ref, s2_ref, t2_ref, o_ref):
    acc = jnp.dot(xr_ref[...], w1a_ref[...], preferred_element_type=jnp.float32, precision=jax.lax.Precision.HIGHEST)
    acc = acc + jnp.dot(xc_ref[...], w1b_ref[...], preferred_element_type=jnp.float32, precision=jax.lax.Precision.HIGHEST)
    h = jnp.maximum(acc * s1_ref[...] + t1_ref[...], 0.0)
    h2 = jnp.dot(h, w2_ref[...], preferred_element_type=jnp.float32, precision=jax.lax.Precision.HIGHEST)
    o_ref[...] = jnp.maximum(h2 * s2_ref[...] + t2_ref[...], 0.0)


def _edge_mlp(xr, xc, w1aT, w1bT, s1, t1, w2T, s2, t2):
    full = lambda i: (0, 0)
    return pl.pallas_call(
        _edge_mlp_body,
        grid=(E // EBLK,),
        in_specs=[
            pl.BlockSpec((EBLK, HIDDEN), lambda i: (i, 0)),
            pl.BlockSpec((EBLK, HIDDEN), lambda i: (i, 0)),
            pl.BlockSpec((HIDDEN, HIDDEN), full),
            pl.BlockSpec((HIDDEN, HIDDEN), full),
            pl.BlockSpec((1, HIDDEN), full),
            pl.BlockSpec((1, HIDDEN), full),
            pl.BlockSpec((HIDDEN, HIDDEN), full),
            pl.BlockSpec((1, HIDDEN), full),
            pl.BlockSpec((1, HIDDEN), full),
        ],
        out_specs=pl.BlockSpec((EBLK, HIDDEN), lambda i: (i, 0)),
        out_shape=jax.ShapeDtypeStruct((E, HIDDEN), jnp.float32),
    )(xr, xc, w1aT, w1bT, s1, t1, w2T, s2, t2)


def _ln1_body(y_ref, r_ref, g_ref, b_ref, o_ref):
    z = y_ref[...] + r_ref[...]
    o_ref[...] = _ln_expr(z, g_ref[...], b_ref[...])


def _res_ln(y, r, g, b):
    full = lambda i: (0, 0)
    return pl.pallas_call(
        _ln1_body,
        grid=(N // NBLK,),
        in_specs=[
            pl.BlockSpec((NBLK, HIDDEN), lambda i: (i, 0)),
            pl.BlockSpec((NBLK, HIDDEN), lambda i: (i, 0)),
            pl.BlockSpec((1, HIDDEN), full),
            pl.BlockSpec((1, HIDDEN), full),
        ],
        out_specs=pl.BlockSpec((NBLK, HIDDEN), lambda i: (i, 0)),
        out_shape=jax.ShapeDtypeStruct((N, HIDDEN), jnp.float32),
    )(y, r, g, b)


def _gatproj_body(x_ref, w_ref, m_ref, h_ref, aa_ref, cm_ref):
    i = pl.program_id(0)
    h = jnp.dot(x_ref[...], w_ref[...], preferred_element_type=jnp.float32, precision=jax.lax.Precision.HIGHEST)
    h_ref[...] = h
    aa = jnp.dot(h, m_ref[...], preferred_element_type=jnp.float32, precision=jax.lax.Precision.HIGHEST)
    aa_ref[...] = aa
    bm = jnp.max(aa, axis=0, keepdims=True)

    @pl.when(i == 0)
    def _():
        cm_ref[...] = bm

    @pl.when(i > 0)
    def _():
        cm_ref[...] = jnp.maximum(cm_ref[...], bm)


def _gat_proj(x, wT, mT):
    full = lambda i: (0, 0)
    return pl.pallas_call(
        _gatproj_body,
        grid=(N // NBLK,),
        in_specs=[
            pl.BlockSpec((NBLK, HIDDEN), lambda i: (i, 0)),
            pl.BlockSpec((HIDDEN, HIDDEN), full),
            pl.BlockSpec((HIDDEN, 2 * HEADS), full),
        ],
        out_specs=[
            pl.BlockSpec((NBLK, HIDDEN), lambda i: (i, 0)),
            pl.BlockSpec((NBLK, 2 * HEADS), lambda i: (i, 0)),
            pl.BlockSpec((1, 2 * HEADS), full),
        ],
        out_shape=[
            jax.ShapeDtypeStruct((N, HIDDEN), jnp.float32),
            jax.ShapeDtypeStruct((N, 2 * HEADS), jnp.float32),
            jax.ShapeDtypeStruct((1, 2 * HEADS), jnp.float32),
        ],
    )(x, wT, mT)


def _attnmsg_body(ar_ref, ac_ref, c_ref, h_ref, r_ref, ex_ref, msg_ref):
    z = ar_ref[:, :HEADS] + ac_ref[:, HEADS:]
    a = jnp.where(z >= 0.0, z, 0.2 * z)
    ex = jnp.exp(a - c_ref[...])
    ex_ref[...] = ex
    ex64 = jnp.dot(ex, r_ref[...], preferred_element_type=jnp.float32, precision=jax.lax.Precision.HIGHEST)
    msg_ref[...] = h_ref[...] * ex64


def _attn_msg(ar, ac, c, h_src, rexp):
    full = lambda i: (0, 0)
    return pl.pallas_call(
        _attnmsg_body,
        grid=(E // EBLK,),
        in_specs=[
            pl.BlockSpec((EBLK, 2 * HEADS), lambda i: (i, 0)),
            pl.BlockSpec((EBLK, 2 * HEADS), lambda i: (i, 0)),
            pl.BlockSpec((1, HEADS), full),
            pl.BlockSpec((EBLK, HIDDEN), lambda i: (i, 0)),
            pl.BlockSpec((HEADS, HIDDEN), full),
        ],
        out_specs=[
            pl.BlockSpec((EBLK, HEADS), lambda i: (i, 0)),
            pl.BlockSpec((EBLK, HIDDEN), lambda i: (i, 0)),
        ],
        out_shape=[
            jax.ShapeDtypeStruct((E, HEADS), jnp.float32),
            jax.ShapeDtypeStruct((E, HIDDEN), jnp.float32),
        ],
    )(ar, ac, c, h_src, rexp)


def _ln2_body(num_ref, den_ref, r_ref, xp_ref, bias_ref,
              g1_ref, b1_ref, g2_ref, b2_ref, o_ref):
    dd = jnp.dot(den_ref[...], r_ref[...], preferred_element_type=jnp.float32, precision=jax.lax.Precision.HIGHEST)
    xg = num_ref[...] / (dd + 1e-16)
    t = _ln_expr(xg + bias_ref[...] + xp_ref[...], g1_ref[...], b1_ref[...])
    o_ref[...] = _ln_expr(t + xp_ref[...], g2_ref[...], b2_ref[...])


def _gat_epilogue(num, den, rexp, xp, bias, g1, b1, g2, b2):
    full = lambda i: (0, 0)
    return pl.pallas_call(
        _ln2_body,
        grid=(N // NBLK,),
        in_specs=[
            pl.BlockSpec((NBLK, HIDDEN), lambda i: (i, 0)),
            pl.BlockSpec((NBLK, HEADS), lambda i: (i, 0)),
            pl.BlockSpec((HEADS, HIDDEN), full),
            pl.BlockSpec((NBLK, HIDDEN), lambda i: (i, 0)),
            pl.BlockSpec((1, HIDDEN), full),
            pl.BlockSpec((1, HIDDEN), full),
            pl.BlockSpec((1, HIDDEN), full),
            pl.BlockSpec((1, HIDDEN), full),
            pl.BlockSpec((1, HIDDEN), full),
        ],
        out_specs=pl.BlockSpec((NBLK, HIDDEN), lambda i: (i, 0)),
        out_shape=jax.ShapeDtypeStruct((N, HIDDEN), jnp.float32),
    )(num, den, rexp, xp, bias, g1, b1, g2, b2)


def _pool_body(x_ref, b_ref, sum_ref, cnt_ref, max_ref):
    i = pl.program_id(0)
    x = x_ref[...]
    bb = b_ref[...]
    gids = jax.lax.broadcasted_iota(jnp.int32, (1, NUM_GRAPHS), 1)
    onehot = (bb == gids).astype(jnp.float32)
    psum = jax.lax.dot_general(onehot, x, (((0,), (0,)), ((), ())),
                               preferred_element_type=jnp.float32, precision=jax.lax.Precision.HIGHEST)
    ones = jnp.ones((x.shape[0], 1), jnp.float32)
    pcnt = jax.lax.dot_general(onehot, ones, (((0,), (0,)), ((), ())),
                               preferred_element_type=jnp.float32, precision=jax.lax.Precision.HIGHEST)
    neg = jnp.float32(-3.0e38)
    rows = []
    for g in range(NUM_GRAPHS):
        mask = bb == g
        rows.append(jnp.max(jnp.where(mask, x, neg), axis=0, keepdims=True))
    pmax = jnp.concatenate(rows, axis=0)

    @pl.when(i == 0)
    def _():
        sum_ref[...] = psum
        cnt_ref[...] = pcnt
        max_ref[...] = pmax

    @pl.when(i > 0)
    def _():
        sum_ref[...] = sum_ref[...] + psum
        cnt_ref[...] = cnt_ref[...] + pcnt
        max_ref[...] = jnp.maximum(max_ref[...], pmax)


def _pool(x, batch2d):
    full = lambda i: (0, 0)
    return pl.pallas_call(
        _pool_body,
        grid=(N // NBLK,),
        in_specs=[
            pl.BlockSpec((NBLK, HIDDEN), lambda i: (i, 0)),
            pl.BlockSpec((NBLK, 1), lambda i: (i, 0)),
        ],
        out_specs=[
            pl.BlockSpec((NUM_GRAPHS, HIDDEN), full),
            pl.BlockSpec((NUM_GRAPHS, 1), full),
            pl.BlockSpec((NUM_GRAPHS, HIDDEN), full),
        ],
        out_shape=[
            jax.ShapeDtypeStruct((NUM_GRAPHS, HIDDEN), jnp.float32),
            jax.ShapeDtypeStruct((NUM_GRAPHS, 1), jnp.float32),
            jax.ShapeDtypeStruct((NUM_GRAPHS, HIDDEN), jnp.float32),
        ],
    )(x, batch2d)


def _head_body(xsum_ref, cnt_ref, xmax_ref, wpa_ref, wpb_ref, bp_ref,
               wc1_ref, bc1_ref, wc2_ref, bc2_ref,
               we1_ref, be1_ref, we2_ref, be2_ref,
               logits_ref, energy_ref):
    cnt = jnp.maximum(cnt_ref[...], 1.0)
    xmean = xsum_ref[...] / cnt
    xm = xmax_ref[...]
    xm = jnp.where(xm > jnp.float32(-1.0e38), xm, 0.0)
    pool = jnp.dot(xmean, wpa_ref[...], preferred_element_type=jnp.float32, precision=jax.lax.Precision.HIGHEST)
    pool = pool + jnp.dot(xm, wpb_ref[...], preferred_element_type=jnp.float32, precision=jax.lax.Precision.HIGHEST)
    pool = jnp.maximum(pool + bp_ref[...], 0.0)
    h1 = jnp.maximum(
        jnp.dot(pool, wc1_ref[...], preferred_element_type=jnp.float32, precision=jax.lax.Precision.HIGHEST) + bc1_ref[...], 0.0)
    logits_ref[...] = jnp.dot(h1, wc2_ref[...], preferred_element_type=jnp.float32, precision=jax.lax.Precision.HIGHEST) + bc2_ref[...]
    e1 = jnp.maximum(
        jnp.dot(pool, we1_ref[...], preferred_element_type=jnp.float32, precision=jax.lax.Precision.HIGHEST) + be1_ref[...], 0.0)
    z = jnp.dot(e1, we2_ref[...], preferred_element_type=jnp.float32, precision=jax.lax.Precision.HIGHEST) + be2_ref[...]
    energy_ref[...] = jnp.maximum(z, 0.0) + jnp.log(1.0 + jnp.exp(-jnp.abs(z)))


def _heads(xsum, cnt, xmax, p):
    wpa = p["pool"]["W"][:, :HIDDEN].T
    wpb = p["pool"]["W"][:, HIDDEN:].T
    bp = p["pool"]["b"].reshape(1, -1)
    wc1 = p["cls1"]["W"].T
    bc1 = p["cls1"]["b"].reshape(1, -1)
    wc2 = p["cls2"]["W"].T
    bc2 = p["cls2"]["b"].reshape(1, -1)
    we1 = p["en1"]["W"].T
    be1 = p["en1"]["b"].reshape(1, -1)
    we2 = p["en2"]["W"].T
    be2 = p["en2"]["b"].reshape(1, -1)
    return pl.pallas_call(
        _head_body,
        out_shape=[
            jax.ShapeDtypeStruct((NUM_GRAPHS, 5), jnp.float32),
            jax.ShapeDtypeStruct((NUM_GRAPHS, 1), jnp.float32),
        ],
    )(xsum, cnt, xmax, wpa, wpb, bp, wc1, bc1, wc2, bc2, we1, be1, we2, be2)


def kernel(x, edge_index, batch, params):
    row = edge_index[0]
    col = edge_index[1]

    encw = params["enc_lin"]["W"].T
    s0, t0 = _bn_fold(params["enc_bn"])
    t0 = t0 + params["enc_lin"]["b"].reshape(1, -1) * s0
    h = _encoder(x, encw, s0, t0)

    for i in range(NUM_BLOCKS):
        pc = params["edge_convs"][i]
        w1 = pc["lin1"]["W"]
        w1aT = w1[:, :HIDDEN].T
        w1bT = w1[:, HIDDEN:].T
        s1, t1 = _bn_fold(pc["bn1"])
        t1 = t1 + pc["lin1"]["b"].reshape(1, -1) * s1
        w2T = pc["lin2"]["W"].T
        s2, t2 = _bn_fold(pc["bn2"])
        t2 = t2 + pc["lin2"]["b"].reshape(1, -1) * s2

        xr, xc = _sc_gather_multi([(h, row), (h, col)])
        he = _edge_mlp(xr, xc, w1aT, w1bT, s1, t1, w2T, s2, t2)
        ec = jax.ops.segment_sum(he, row, num_segments=N)
        ln = params["lns"][2 * i]
        x1 = _res_ln(ec, h, ln["g"].reshape(1, -1), ln["b"].reshape(1, -1))

        g = params["gats"][i]
        idx_h = jnp.repeat(jnp.arange(HEADS), HEAD_DIM)
        ms = jnp.zeros((HEADS, HIDDEN), jnp.float32)
        ms = ms.at[idx_h, jnp.arange(HIDDEN)].set(g["att_src"].reshape(-1))
        md = jnp.zeros((HEADS, HIDDEN), jnp.float32)
        md = md.at[idx_h, jnp.arange(HIDDEN)].set(g["att_dst"].reshape(-1))
        mT = jnp.concatenate([ms, md], axis=0).T
        hh, aa, cm = _gat_proj(x1, g["W"].T, mT)
        c = cm[:, :HEADS] + cm[:, HEADS:]
        ar, ac, h_src = _sc_gather_multi([(aa, row), (aa, col), (hh, row)])
        rexp = jnp.zeros((HEADS, HIDDEN), jnp.float32)
        rexp = rexp.at[idx_h, jnp.arange(HIDDEN)].set(1.0)
        ex, msg = _attn_msg(ar, ac, c, h_src, rexp)
        denom = jax.ops.segment_sum(ex, col, num_segments=N)
        num = jax.ops.segment_sum(msg, col, num_segments=N)
        ln2 = params["lns"][2 * i + 1]
        h = _gat_epilogue(num, denom, rexp, x1, g["bias"].reshape(1, -1),
                          g["ln_g"].reshape(1, -1), g["ln_b"].reshape(1, -1),
                          ln2["g"].reshape(1, -1), ln2["b"].reshape(1, -1))

    xsum, cnt, xmax = _pool(h, batch.reshape(-1, 1).astype(jnp.int32))
    logits, energy = _heads(xsum, cnt, xmax, params)
    return logits, energy


# speedup vs baseline: 15.2299x; 1.2028x over previous
import functools

import jax
import jax.numpy as jnp
import numpy as np
from jax import lax
from jax.experimental import pallas as pl
from jax.experimental.pallas import tpu as pltpu
from jax.experimental.pallas import tpu_sc as plsc

N = 50000
E = 800000
HIDDEN = 64
HEADS = 8
HEAD_DIM = 8
NUM_GRAPHS = 64
NUM_BLOCKS = 3

EBLK = 8000
NBLK = 5000


def _bn_fold(bn):
    s = bn["g"] / jnp.sqrt(bn["rv"] + 1e-5)
    t = bn["b"] - bn["rm"] * s
    return s.reshape(1, -1), t.reshape(1, -1)


def _ln_expr(z, g, b):
    m = jnp.mean(z, axis=-1, keepdims=True)
    v = jnp.mean((z - m) ** 2, axis=-1, keepdims=True)
    return (z - m) * jax.lax.rsqrt(v + 1e-5) * g + b


NWORKERS = 32
EPW = E // NWORKERS
G_UNIT = 128
G_GROUP = 512
G_NGRP = EPW // G_GROUP
G_REM = EPW - G_NGRP * G_GROUP
G_REMU = G_REM // G_UNIT
G_TAIL = G_REM - G_REMU * G_UNIT


def _sc_gather_multi(pairs):
    dims = tuple(int(t.shape[1]) for t, _ in pairs)
    np_ = len(pairs)
    udims = sorted(set(dims))

    scratch = [pltpu.VMEM((EPW,), jnp.int32)]
    for d in udims:
        scratch += [pltpu.VMEM((G_GROUP, d), jnp.float32)] * 2
    scratch += [pltpu.SemaphoreType.DMA] * 4

    @functools.partial(
        pl.kernel,
        mesh=plsc.VectorSubcoreMesh(core_axis_name="c", subcore_axis_name="s"),
        out_type=[jax.ShapeDtypeStruct((E, d), jnp.float32) for d in dims],
        scratch_types=scratch,
        compiler_params=pltpu.CompilerParams(use_tc_tiling_on_sc=False),
    )
    def k(*refs):
        tables = refs[:np_]
        idxs = refs[np_:2 * np_]
        outs = refs[2 * np_:3 * np_]
        idxb = refs[3 * np_]
        pos = 3 * np_ + 1
        bufmap = {}
        for d in udims:
            bufmap[d] = (refs[pos], refs[pos + 1])
            pos += 2
        gs = refs[pos:pos + 2]
        ws = refs[pos + 2:pos + 4]
        w = lax.axis_index("s") * 2 + lax.axis_index("c")
        base = w * EPW

        for p in range(np_):
            tab, idx, out = tables[p], idxs[p], outs[p]
            rb = bufmap[dims[p]]
            pltpu.sync_copy(idx.at[pl.ds(base, EPW)], idxb)

            def grp_dma(grp, b, issue):
                for k_ in range(4):
                    off = grp * G_GROUP + k_ * G_UNIT
                    c = pltpu.make_async_copy(
                        tab.at[idxb.at[pl.ds(off, G_UNIT)]],
                        rb[b].at[pl.ds(k_ * G_UNIT, G_UNIT)], gs[b])
                    c.start() if issue else c.wait()

            def body(jj, carry):
                for b in range(2):
                    grp_dma(2 * jj + b, b, True)
                for b in range(2):
                    grp = 2 * jj + b
                    grp_dma(grp, b, False)
                    pltpu.async_copy(
                        rb[b], out.at[pl.ds(base + grp * G_GROUP, G_GROUP)], ws[b])
                for b in range(2):
                    grp = 2 * jj + b
                    pltpu.make_async_copy(
                        rb[b], out.at[pl.ds(base + grp * G_GROUP, G_GROUP)], ws[b]).wait()
                return carry

            lax.fori_loop(0, G_NGRP // 2, body, 0)

            rem_base = G_NGRP * G_GROUP
            for k_ in range(G_REMU):
                pltpu.async_copy(
                    tab.at[idxb.at[pl.ds(rem_base + k_ * G_UNIT, G_UNIT)]],
                    rb[0].at[pl.ds(k_ * G_UNIT, G_UNIT)], gs[0])
            pltpu.async_copy(
                tab.at[idxb.at[pl.ds(rem_base + G_REMU * G_UNIT, G_TAIL)]],
                rb[0].at[pl.ds(G_REMU * G_UNIT, G_TAIL)], gs[0])
            for k_ in range(G_REMU):
                pltpu.make_async_copy(
                    tab.at[idxb.at[pl.ds(rem_base + k_ * G_UNIT, G_UNIT)]],
                    rb[0].at[pl.ds(k_ * G_UNIT, G_UNIT)], gs[0]).wait()
            pltpu.make_async_copy(
                tab.at[idxb.at[pl.ds(rem_base + G_REMU * G_UNIT, G_TAIL)]],
                rb[0].at[pl.ds(G_REMU * G_UNIT, G_TAIL)], gs[0]).wait()
            pltpu.async_copy(
                rb[0].at[pl.ds(0, G_REM)], out.at[pl.ds(base + rem_base, G_REM)], ws[0])
            pltpu.make_async_copy(
                rb[0].at[pl.ds(0, G_REM)], out.at[pl.ds(base + rem_base, G_REM)], ws[0]).wait()

    res = k(*[t for t, _ in pairs], *[i for _, i in pairs])
    return list(res) if np_ > 1 else [res]


def _enc_body(x_ref, w_ref, s_ref, t_ref, o_ref):
    y = jnp.dot(x_ref[...], w_ref[...], preferred_element_type=jnp.float32, precision=jax.lax.Precision.HIGHEST)
    o_ref[...] = jnp.maximum(y * s_ref[...] + t_ref[...], 0.0)


def _encoder(x, wT, s, t):
    return pl.pallas_call(
        _enc_body,
        grid=(N // NBLK,),
        in_specs=[
            pl.BlockSpec((NBLK, 4), lambda i: (i, 0)),
            pl.BlockSpec((4, HIDDEN), lambda i: (0, 0)),
            pl.BlockSpec((1, HIDDEN), lambda i: (0, 0)),
            pl.BlockSpec((1, HIDDEN), lambda i: (0, 0)),
        ],
        out_specs=pl.BlockSpec((NBLK, HIDDEN), lambda i: (i, 0)),
        out_shape=jax.ShapeDtypeStruct((N, HIDDEN), jnp.float32),
    )(x, wT, s, t)


def _edge_mlp_body(xr_ref, xc_ref, w1a_ref, w1b_ref, s1_ref, t1_ref,
                   w2_ref, s2_ref, t2_ref, o_ref):
    acc = jnp.dot(xr_ref[...], w1a_ref[...], preferred_element_type=jnp.float32, precision=jax.lax.Precision.HIGHEST)
    acc = acc + jnp.dot(xc_ref[...], w1b_ref[...], preferred_element_type=jnp.float32, precision=jax.lax.Precision.HIGHEST)
    h = jnp.maximum(acc * s1_ref[...] + t1_ref[...], 0.0)
    h2 = jnp.dot(h, w2_ref[...], preferred_element_type=jnp.float32, precision=jax.lax.Precision.HIGHEST)
    o_ref[...] = jnp.maximum(h2 * s2_ref[...] + t2_ref[...], 0.0)


def _edge_mlp(xr, xc, w1aT, w1bT, s1, t1, w2T, s2, t2):
    full = lambda i: (0, 0)
    return pl.pallas_call(
        _edge_mlp_body,
        grid=(E // EBLK,),
        in_specs=[
            pl.BlockSpec((EBLK, HIDDEN), lambda i: (i, 0)),
            pl.BlockSpec((EBLK, HIDDEN), lambda i: (i, 0)),
            pl.BlockSpec((HIDDEN, HIDDEN), full),
            pl.BlockSpec((HIDDEN, HIDDEN), full),
            pl.BlockSpec((1, HIDDEN), full),
            pl.BlockSpec((1, HIDDEN), full),
            pl.BlockSpec((HIDDEN, HIDDEN), full),
            pl.BlockSpec((1, HIDDEN), full),
            pl.BlockSpec((1, HIDDEN), full),
        ],
        out_specs=pl.BlockSpec((EBLK, HIDDEN), lambda i: (i, 0)),
        out_shape=jax.ShapeDtypeStruct((E, HIDDEN), jnp.float32),
    )(xr, xc, w1aT, w1bT, s1, t1, w2T, s2, t2)


def _ln1_body(y_ref, r_ref, g_ref, b_ref, o_ref):
    z = y_ref[...] + r_ref[...]
    o_ref[...] = _ln_expr(z, g_ref[...], b_ref[...])


def _res_ln(y, r, g, b):
    full = lambda i: (0, 0)
    return pl.pallas_call(
        _ln1_body,
        grid=(N // NBLK,),
        in_specs=[
            pl.BlockSpec((NBLK, HIDDEN), lambda i: (i, 0)),
            pl.BlockSpec((NBLK, HIDDEN), lambda i: (i, 0)),
            pl.BlockSpec((1, HIDDEN), full),
            pl.BlockSpec((1, HIDDEN), full),
        ],
        out_specs=pl.BlockSpec((NBLK, HIDDEN), lambda i: (i, 0)),
        out_shape=jax.ShapeDtypeStruct((N, HIDDEN), jnp.float32),
    )(y, r, g, b)


def _gatproj_body(x_ref, w_ref, m_ref, h_ref, aa_ref, cm_ref):
    i = pl.program_id(0)
    h = jnp.dot(x_ref[...], w_ref[...], preferred_element_type=jnp.float32, precision=jax.lax.Precision.HIGHEST)
    h_ref[...] = h
    aa = jnp.dot(h, m_ref[...], preferred_element_type=jnp.float32, precision=jax.lax.Precision.HIGHEST)
    aa_ref[...] = aa
    bm = jnp.max(aa, axis=0, keepdims=True)

    @pl.when(i == 0)
    def _():
        cm_ref[...] = bm

    @pl.when(i > 0)
    def _():
        cm_ref[...] = jnp.maximum(cm_ref[...], bm)


def _gat_proj(x, wT, mT):
    full = lambda i: (0, 0)
    return pl.pallas_call(
        _gatproj_body,
        grid=(N // NBLK,),
        in_specs=[
            pl.BlockSpec((NBLK, HIDDEN), lambda i: (i, 0)),
            pl.BlockSpec((HIDDEN, HIDDEN), full),
            pl.BlockSpec((HIDDEN, 2 * HEADS), full),
        ],
        out_specs=[
            pl.BlockSpec((NBLK, HIDDEN), lambda i: (i, 0)),
            pl.BlockSpec((NBLK, 2 * HEADS), lambda i: (i, 0)),
            pl.BlockSpec((1, 2 * HEADS), full),
        ],
        out_shape=[
            jax.ShapeDtypeStruct((N, HIDDEN), jnp.float32),
            jax.ShapeDtypeStruct((N, 2 * HEADS), jnp.float32),
            jax.ShapeDtypeStruct((1, 2 * HEADS), jnp.float32),
        ],
    )(x, wT, mT)


def _attnmsg_body(ar_ref, ac_ref, c_ref, h_ref, r_ref, y_ref):
    z = ar_ref[:, :HEADS] + ac_ref[:, HEADS:]
    a = jnp.where(z >= 0.0, z, 0.2 * z)
    ex = jnp.exp(a - c_ref[...])
    ex64 = jnp.dot(ex, r_ref[...], preferred_element_type=jnp.float32, precision=jax.lax.Precision.HIGHEST)
    y_ref[:, :HIDDEN] = h_ref[...] * ex64
    y_ref[:, HIDDEN:] = ex


def _attn_msg(ar, ac, c, h_src, rexp):
    full = lambda i: (0, 0)
    return pl.pallas_call(
        _attnmsg_body,
        grid=(E // EBLK,),
        in_specs=[
            pl.BlockSpec((EBLK, 2 * HEADS), lambda i: (i, 0)),
            pl.BlockSpec((EBLK, 2 * HEADS), lambda i: (i, 0)),
            pl.BlockSpec((1, HEADS), full),
            pl.BlockSpec((EBLK, HIDDEN), lambda i: (i, 0)),
            pl.BlockSpec((HEADS, HIDDEN), full),
        ],
        out_specs=pl.BlockSpec((EBLK, HIDDEN + HEADS), lambda i: (i, 0)),
        out_shape=jax.ShapeDtypeStruct((E, HIDDEN + HEADS), jnp.float32),
    )(ar, ac, c, h_src, rexp)


def _ln2_body(s_ref, r_ref, xp_ref, bias_ref,
              g1_ref, b1_ref, g2_ref, b2_ref, o_ref):
    dd = jnp.dot(s_ref[:, HIDDEN:], r_ref[...], preferred_element_type=jnp.float32, precision=jax.lax.Precision.HIGHEST)
    xg = s_ref[:, :HIDDEN] / (dd + 1e-16)
    t = _ln_expr(xg + bias_ref[...] + xp_ref[...], g1_ref[...], b1_ref[...])
    o_ref[...] = _ln_expr(t + xp_ref[...], g2_ref[...], b2_ref[...])


def _gat_epilogue(s, rexp, xp, bias, g1, b1, g2, b2):
    full = lambda i: (0, 0)
    return pl.pallas_call(
        _ln2_body,
        grid=(N // NBLK,),
        in_specs=[
            pl.BlockSpec((NBLK, HIDDEN + HEADS), lambda i: (i, 0)),
            pl.BlockSpec((HEADS, HIDDEN), full),
            pl.BlockSpec((NBLK, HIDDEN), lambda i: (i, 0)),
            pl.BlockSpec((1, HIDDEN), full),
            pl.BlockSpec((1, HIDDEN), full),
            pl.BlockSpec((1, HIDDEN), full),
            pl.BlockSpec((1, HIDDEN), full),
            pl.BlockSpec((1, HIDDEN), full),
        ],
        out_specs=pl.BlockSpec((NBLK, HIDDEN), lambda i: (i, 0)),
        out_shape=jax.ShapeDtypeStruct((N, HIDDEN), jnp.float32),
    )(s, rexp, xp, bias, g1, b1, g2, b2)


def _pool_body(x_ref, b_ref, sum_ref, cnt_ref, max_ref):
    i = pl.program_id(0)
    x = x_ref[...]
    bb = b_ref[...]
    gids = jax.lax.broadcasted_iota(jnp.int32, (1, NUM_GRAPHS), 1)
    onehot = (bb == gids).astype(jnp.float32)
    psum = jax.lax.dot_general(onehot, x, (((0,), (0,)), ((), ())),
                               preferred_element_type=jnp.float32, precision=jax.lax.Precision.HIGHEST)
    ones = jnp.ones((x.shape[0], 1), jnp.float32)
    pcnt = jax.lax.dot_general(onehot, ones, (((0,), (0,)), ((), ())),
                               preferred_element_type=jnp.float32, precision=jax.lax.Precision.HIGHEST)
    neg = jnp.float32(-3.0e38)
    rows = []
    for g in range(NUM_GRAPHS):
        mask = bb == g
        rows.append(jnp.max(jnp.where(mask, x, neg), axis=0, keepdims=True))
    pmax = jnp.concatenate(rows, axis=0)

    @pl.when(i == 0)
    def _():
        sum_ref[...] = psum
        cnt_ref[...] = pcnt
        max_ref[...] = pmax

    @pl.when(i > 0)
    def _():
        sum_ref[...] = sum_ref[...] + psum
        cnt_ref[...] = cnt_ref[...] + pcnt
        max_ref[...] = jnp.maximum(max_ref[...], pmax)


def _pool(x, batch2d):
    full = lambda i: (0, 0)
    return pl.pallas_call(
        _pool_body,
        grid=(N // NBLK,),
        in_specs=[
            pl.BlockSpec((NBLK, HIDDEN), lambda i: (i, 0)),
            pl.BlockSpec((NBLK, 1), lambda i: (i, 0)),
        ],
        out_specs=[
            pl.BlockSpec((NUM_GRAPHS, HIDDEN), full),
            pl.BlockSpec((NUM_GRAPHS, 1), full),
            pl.BlockSpec((NUM_GRAPHS, HIDDEN), full),
        ],
        out_shape=[
            jax.ShapeDtypeStruct((NUM_GRAPHS, HIDDEN), jnp.float32),
            jax.ShapeDtypeStruct((NUM_GRAPHS, 1), jnp.float32),
            jax.ShapeDtypeStruct((NUM_GRAPHS, HIDDEN), jnp.float32),
        ],
    )(x, batch2d)


def _head_body(xsum_ref, cnt_ref, xmax_ref, wpa_ref, wpb_ref, bp_ref,
               wc1_ref, bc1_ref, wc2_ref, bc2_ref,
               we1_ref, be1_ref, we2_ref, be2_ref,
               logits_ref, energy_ref):
    cnt = jnp.maximum(cnt_ref[...], 1.0)
    xmean = xsum_ref[...] / cnt
    xm = xmax_ref[...]
    xm = jnp.where(xm > jnp.float32(-1.0e38), xm, 0.0)
    pool = jnp.dot(xmean, wpa_ref[...], preferred_element_type=jnp.float32, precision=jax.lax.Precision.HIGHEST)
    pool = pool + jnp.dot(xm, wpb_ref[...], preferred_element_type=jnp.float32, precision=jax.lax.Precision.HIGHEST)
    pool = jnp.maximum(pool + bp_ref[...], 0.0)
    h1 = jnp.maximum(
        jnp.dot(pool, wc1_ref[...], preferred_element_type=jnp.float32, precision=jax.lax.Precision.HIGHEST) + bc1_ref[...], 0.0)
    logits_ref[...] = jnp.dot(h1, wc2_ref[...], preferred_element_type=jnp.float32, precision=jax.lax.Precision.HIGHEST) + bc2_ref[...]
    e1 = jnp.maximum(
        jnp.dot(pool, we1_ref[...], preferred_element_type=jnp.float32, precision=jax.lax.Precision.HIGHEST) + be1_ref[...], 0.0)
    z = jnp.dot(e1, we2_ref[...], preferred_element_type=jnp.float32, precision=jax.lax.Precision.HIGHEST) + be2_ref[...]
    energy_ref[...] = jnp.maximum(z, 0.0) + jnp.log(1.0 + jnp.exp(-jnp.abs(z)))


def _heads(xsum, cnt, xmax, p):
    wpa = p["pool"]["W"][:, :HIDDEN].T
    wpb = p["pool"]["W"][:, HIDDEN:].T
    bp = p["pool"]["b"].reshape(1, -1)
    wc1 = p["cls1"]["W"].T
    bc1 = p["cls1"]["b"].reshape(1, -1)
    wc2 = p["cls2"]["W"].T
    bc2 = p["cls2"]["b"].reshape(1, -1)
    we1 = p["en1"]["W"].T
    be1 = p["en1"]["b"].reshape(1, -1)
    we2 = p["en2"]["W"].T
    be2 = p["en2"]["b"].reshape(1, -1)
    return pl.pallas_call(
        _head_body,
        out_shape=[
            jax.ShapeDtypeStruct((NUM_GRAPHS, 5), jnp.float32),
            jax.ShapeDtypeStruct((NUM_GRAPHS, 1), jnp.float32),
        ],
    )(xsum, cnt, xmax, wpa, wpb, bp, wc1, bc1, wc2, bc2, we1, be1, we2, be2)


def kernel(x, edge_index, batch, params):
    row = edge_index[0]
    col = edge_index[1]

    encw = params["enc_lin"]["W"].T
    s0, t0 = _bn_fold(params["enc_bn"])
    t0 = t0 + params["enc_lin"]["b"].reshape(1, -1) * s0
    h = _encoder(x, encw, s0, t0)

    for i in range(NUM_BLOCKS):
        pc = params["edge_convs"][i]
        w1 = pc["lin1"]["W"]
        w1aT = w1[:, :HIDDEN].T
        w1bT = w1[:, HIDDEN:].T
        s1, t1 = _bn_fold(pc["bn1"])
        t1 = t1 + pc["lin1"]["b"].reshape(1, -1) * s1
        w2T = pc["lin2"]["W"].T
        s2, t2 = _bn_fold(pc["bn2"])
        t2 = t2 + pc["lin2"]["b"].reshape(1, -1) * s2

        xr, xc = _sc_gather_multi([(h, row), (h, col)])
        he = _edge_mlp(xr, xc, w1aT, w1bT, s1, t1, w2T, s2, t2)
        ec = jax.ops.segment_sum(he, row, num_segments=N)
        ln = params["lns"][2 * i]
        x1 = _res_ln(ec, h, ln["g"].reshape(1, -1), ln["b"].reshape(1, -1))

        g = params["gats"][i]
        idx_h = jnp.repeat(jnp.arange(HEADS), HEAD_DIM)
        ms = jnp.zeros((HEADS, HIDDEN), jnp.float32)
        ms = ms.at[idx_h, jnp.arange(HIDDEN)].set(g["att_src"].reshape(-1))
        md = jnp.zeros((HEADS, HIDDEN), jnp.float32)
        md = md.at[idx_h, jnp.arange(HIDDEN)].set(g["att_dst"].reshape(-1))
        mT = jnp.concatenate([ms, md], axis=0).T
        hh, aa, cm = _gat_proj(x1, g["W"].T, mT)
        c = cm[:, :HEADS] + cm[:, HEADS:]
        ar, ac, h_src = _sc_gather_multi([(aa, row), (aa, col), (hh, row)])
        rexp = jnp.zeros((HEADS, HIDDEN), jnp.float32)
        rexp = rexp.at[idx_h, jnp.arange(HIDDEN)].set(1.0)
        y = _attn_msg(ar, ac, c, h_src, rexp)
        s = jax.ops.segment_sum(y, col, num_segments=N)
        ln2 = params["lns"][2 * i + 1]
        h = _gat_epilogue(s, rexp, x1, g["bias"].reshape(1, -1),
                          g["ln_g"].reshape(1, -1), g["ln_b"].reshape(1, -1),
                          ln2["g"].reshape(1, -1), ln2["b"].reshape(1, -1))

    xsum, cnt, xmax = _pool(h, batch.reshape(-1, 1).astype(jnp.int32))
    logits, energy = _heads(xsum, cnt, xmax, params)
    return logits, energy


# speedup vs baseline: 15.6715x; 1.0290x over previous
import functools

import jax
import jax.numpy as jnp
import numpy as np
from jax import lax
from jax.experimental import pallas as pl
from jax.experimental.pallas import tpu as pltpu
from jax.experimental.pallas import tpu_sc as plsc

N = 50000
E = 800000
HIDDEN = 64
HEADS = 8
HEAD_DIM = 8
NUM_GRAPHS = 64
NUM_BLOCKS = 3

EBLK = 8000
NBLK = 5000


def _bn_fold(bn):
    s = bn["g"] / jnp.sqrt(bn["rv"] + 1e-5)
    t = bn["b"] - bn["rm"] * s
    return s.reshape(1, -1), t.reshape(1, -1)


def _ln_expr(z, g, b):
    m = jnp.mean(z, axis=-1, keepdims=True)
    v = jnp.mean((z - m) ** 2, axis=-1, keepdims=True)
    return (z - m) * jax.lax.rsqrt(v + 1e-5) * g + b


NWORKERS = 32
EPW = E // NWORKERS
G_UNIT = 128
G_GROUP = 512
G_NGRP = EPW // G_GROUP
G_REM = EPW - G_NGRP * G_GROUP
G_REMU = G_REM // G_UNIT
G_TAIL = G_REM - G_REMU * G_UNIT


def _sc_gather_multi(pairs):
    dims = tuple(int(t.shape[1]) for t, _ in pairs)
    np_ = len(pairs)
    udims = sorted(set(dims))

    scratch = [pltpu.VMEM((EPW,), jnp.int32)]
    for d in udims:
        scratch += [pltpu.VMEM((G_GROUP, d), jnp.float32)] * 2
    scratch += [pltpu.SemaphoreType.DMA] * 4

    @functools.partial(
        pl.kernel,
        mesh=plsc.VectorSubcoreMesh(core_axis_name="c", subcore_axis_name="s"),
        out_type=[jax.ShapeDtypeStruct((E, d), jnp.float32) for d in dims],
        scratch_types=scratch,
        compiler_params=pltpu.CompilerParams(use_tc_tiling_on_sc=False),
    )
    def k(*refs):
        tables = refs[:np_]
        idxs = refs[np_:2 * np_]
        outs = refs[2 * np_:3 * np_]
        idxb = refs[3 * np_]
        pos = 3 * np_ + 1
        bufmap = {}
        for d in udims:
            bufmap[d] = (refs[pos], refs[pos + 1])
            pos += 2
        gs = refs[pos:pos + 2]
        ws = refs[pos + 2:pos + 4]
        w = lax.axis_index("s") * 2 + lax.axis_index("c")
        base = w * EPW

        for p in range(np_):
            tab, idx, out = tables[p], idxs[p], outs[p]
            rb = bufmap[dims[p]]
            pltpu.sync_copy(idx.at[pl.ds(base, EPW)], idxb)

            def grp_dma(grp, b, issue):
                for k_ in range(4):
                    off = grp * G_GROUP + k_ * G_UNIT
                    c = pltpu.make_async_copy(
                        tab.at[idxb.at[pl.ds(off, G_UNIT)]],
                        rb[b].at[pl.ds(k_ * G_UNIT, G_UNIT)], gs[b])
                    c.start() if issue else c.wait()

            def body(jj, carry):
                for b in range(2):
                    grp = 2 * jj + b

                    @pl.when(jj > 0)
                    def _():
                        pltpu.make_async_copy(
                            rb[b], out.at[pl.ds(base + (grp - 2) * G_GROUP, G_GROUP)],
                            ws[b]).wait()
                    grp_dma(grp, b, True)
                for b in range(2):
                    grp = 2 * jj + b
                    grp_dma(grp, b, False)
                    pltpu.async_copy(
                        rb[b], out.at[pl.ds(base + grp * G_GROUP, G_GROUP)], ws[b])
                return carry

            lax.fori_loop(0, G_NGRP // 2, body, 0)
            for b in range(2):
                pltpu.make_async_copy(
                    rb[b], out.at[pl.ds(base + (G_NGRP - 2 + b) * G_GROUP, G_GROUP)],
                    ws[b]).wait()

            rem_base = G_NGRP * G_GROUP
            for k_ in range(G_REMU):
                pltpu.async_copy(
                    tab.at[idxb.at[pl.ds(rem_base + k_ * G_UNIT, G_UNIT)]],
                    rb[0].at[pl.ds(k_ * G_UNIT, G_UNIT)], gs[0])
            pltpu.async_copy(
                tab.at[idxb.at[pl.ds(rem_base + G_REMU * G_UNIT, G_TAIL)]],
                rb[0].at[pl.ds(G_REMU * G_UNIT, G_TAIL)], gs[0])
            for k_ in range(G_REMU):
                pltpu.make_async_copy(
                    tab.at[idxb.at[pl.ds(rem_base + k_ * G_UNIT, G_UNIT)]],
                    rb[0].at[pl.ds(k_ * G_UNIT, G_UNIT)], gs[0]).wait()
            pltpu.make_async_copy(
                tab.at[idxb.at[pl.ds(rem_base + G_REMU * G_UNIT, G_TAIL)]],
                rb[0].at[pl.ds(G_REMU * G_UNIT, G_TAIL)], gs[0]).wait()
            pltpu.async_copy(
                rb[0].at[pl.ds(0, G_REM)], out.at[pl.ds(base + rem_base, G_REM)], ws[0])
            pltpu.make_async_copy(
                rb[0].at[pl.ds(0, G_REM)], out.at[pl.ds(base + rem_base, G_REM)], ws[0]).wait()

    res = k(*[t for t, _ in pairs], *[i for _, i in pairs])
    return list(res) if np_ > 1 else [res]


def _enc_body(x_ref, w_ref, s_ref, t_ref, o_ref):
    y = jnp.dot(x_ref[...], w_ref[...], preferred_element_type=jnp.float32, precision=jax.lax.Precision.HIGHEST)
    o_ref[...] = jnp.maximum(y * s_ref[...] + t_ref[...], 0.0)


def _encoder(x, wT, s, t):
    return pl.pallas_call(
        _enc_body,
        grid=(N // NBLK,),
        in_specs=[
            pl.BlockSpec((NBLK, 4), lambda i: (i, 0)),
            pl.BlockSpec((4, HIDDEN), lambda i: (0, 0)),
            pl.BlockSpec((1, HIDDEN), lambda i: (0, 0)),
            pl.BlockSpec((1, HIDDEN), lambda i: (0, 0)),
        ],
        out_specs=pl.BlockSpec((NBLK, HIDDEN), lambda i: (i, 0)),
        out_shape=jax.ShapeDtypeStruct((N, HIDDEN), jnp.float32),
    )(x, wT, s, t)


def _edge_mlp_body(xr_ref, xc_ref, w1a_ref, w1b_ref, s1_ref, t1_ref,
                   w2_ref, s2_ref, t2_ref, o_ref):
    acc = jnp.dot(xr_ref[...], w1a_ref[...], preferred_element_type=jnp.float32, precision=jax.lax.Precision.HIGHEST)
    acc = acc + jnp.dot(xc_ref[...], w1b_ref[...], preferred_element_type=jnp.float32, precision=jax.lax.Precision.HIGHEST)
    h = jnp.maximum(acc * s1_ref[...] + t1_ref[...], 0.0)
    h2 = jnp.dot(h, w2_ref[...], preferred_element_type=jnp.float32, precision=jax.lax.Precision.HIGHEST)
    o_ref[...] = jnp.maximum(h2 * s2_ref[...] + t2_ref[...], 0.0)


def _edge_mlp(xr, xc, w1aT, w1bT, s1, t1, w2T, s2, t2):
    full = lambda i: (0, 0)
    return pl.pallas_call(
        _edge_mlp_body,
        grid=(E // EBLK,),
        in_specs=[
            pl.BlockSpec((EBLK, HIDDEN), lambda i: (i, 0)),
            pl.BlockSpec((EBLK, HIDDEN), lambda i: (i, 0)),
            pl.BlockSpec((HIDDEN, HIDDEN), full),
            pl.BlockSpec((HIDDEN, HIDDEN), full),
            pl.BlockSpec((1, HIDDEN), full),
            pl.BlockSpec((1, HIDDEN), full),
            pl.BlockSpec((HIDDEN, HIDDEN), full),
            pl.BlockSpec((1, HIDDEN), full),
            pl.BlockSpec((1, HIDDEN), full),
        ],
        out_specs=pl.BlockSpec((EBLK, HIDDEN), lambda i: (i, 0)),
        out_shape=jax.ShapeDtypeStruct((E, HIDDEN), jnp.float32),
    )(xr, xc, w1aT, w1bT, s1, t1, w2T, s2, t2)


def _ln1_body(y_ref, r_ref, g_ref, b_ref, o_ref):
    z = y_ref[...] + r_ref[...]
    o_ref[...] = _ln_expr(z, g_ref[...], b_ref[...])


def _res_ln(y, r, g, b):
    full = lambda i: (0, 0)
    return pl.pallas_call(
        _ln1_body,
        grid=(N // NBLK,),
        in_specs=[
            pl.BlockSpec((NBLK, HIDDEN), lambda i: (i, 0)),
            pl.BlockSpec((NBLK, HIDDEN), lambda i: (i, 0)),
            pl.BlockSpec((1, HIDDEN), full),
            pl.BlockSpec((1, HIDDEN), full),
        ],
        out_specs=pl.BlockSpec((NBLK, HIDDEN), lambda i: (i, 0)),
        out_shape=jax.ShapeDtypeStruct((N, HIDDEN), jnp.float32),
    )(y, r, g, b)


def _gatproj_body(x_ref, w_ref, m_ref, haa_ref, aa_ref, cm_ref):
    i = pl.program_id(0)
    h = jnp.dot(x_ref[...], w_ref[...], preferred_element_type=jnp.float32, precision=jax.lax.Precision.HIGHEST)
    haa_ref[:, :HIDDEN] = h
    aa = jnp.dot(h, m_ref[...], preferred_element_type=jnp.float32, precision=jax.lax.Precision.HIGHEST)
    haa_ref[:, HIDDEN:] = aa
    aa_ref[...] = aa
    bm = jnp.max(aa, axis=0, keepdims=True)

    @pl.when(i == 0)
    def _():
        cm_ref[...] = bm

    @pl.when(i > 0)
    def _():
        cm_ref[...] = jnp.maximum(cm_ref[...], bm)


def _gat_proj(x, wT, mT):
    full = lambda i: (0, 0)
    return pl.pallas_call(
        _gatproj_body,
        grid=(N // NBLK,),
        in_specs=[
            pl.BlockSpec((NBLK, HIDDEN), lambda i: (i, 0)),
            pl.BlockSpec((HIDDEN, HIDDEN), full),
            pl.BlockSpec((HIDDEN, 2 * HEADS), full),
        ],
        out_specs=[
            pl.BlockSpec((NBLK, HIDDEN + 2 * HEADS), lambda i: (i, 0)),
            pl.BlockSpec((NBLK, 2 * HEADS), lambda i: (i, 0)),
            pl.BlockSpec((1, 2 * HEADS), full),
        ],
        out_shape=[
            jax.ShapeDtypeStruct((N, HIDDEN + 2 * HEADS), jnp.float32),
            jax.ShapeDtypeStruct((N, 2 * HEADS), jnp.float32),
            jax.ShapeDtypeStruct((1, 2 * HEADS), jnp.float32),
        ],
    )(x, wT, mT)


def _attnmsg_body(har_ref, ac_ref, c_ref, r_ref, y_ref):
    z = har_ref[:, HIDDEN:HIDDEN + HEADS] + ac_ref[:, HEADS:]
    a = jnp.where(z >= 0.0, z, 0.2 * z)
    ex = jnp.exp(a - c_ref[...])
    ex64 = jnp.dot(ex, r_ref[...], preferred_element_type=jnp.float32, precision=jax.lax.Precision.HIGHEST)
    y_ref[:, :HIDDEN] = har_ref[:, :HIDDEN] * ex64
    y_ref[:, HIDDEN:] = ex


def _attn_msg(har, ac, c, rexp):
    full = lambda i: (0, 0)
    return pl.pallas_call(
        _attnmsg_body,
        grid=(E // EBLK,),
        in_specs=[
            pl.BlockSpec((EBLK, HIDDEN + 2 * HEADS), lambda i: (i, 0)),
            pl.BlockSpec((EBLK, 2 * HEADS), lambda i: (i, 0)),
            pl.BlockSpec((1, HEADS), full),
            pl.BlockSpec((HEADS, HIDDEN), full),
        ],
        out_specs=pl.BlockSpec((EBLK, HIDDEN + HEADS), lambda i: (i, 0)),
        out_shape=jax.ShapeDtypeStruct((E, HIDDEN + HEADS), jnp.float32),
    )(har, ac, c, rexp)


def _ln2_body(s_ref, r_ref, xp_ref, bias_ref,
              g1_ref, b1_ref, g2_ref, b2_ref, o_ref):
    dd = jnp.dot(s_ref[:, HIDDEN:], r_ref[...], preferred_element_type=jnp.float32, precision=jax.lax.Precision.HIGHEST)
    xg = s_ref[:, :HIDDEN] / (dd + 1e-16)
    t = _ln_expr(xg + bias_ref[...] + xp_ref[...], g1_ref[...], b1_ref[...])
    o_ref[...] = _ln_expr(t + xp_ref[...], g2_ref[...], b2_ref[...])


def _gat_epilogue(s, rexp, xp, bias, g1, b1, g2, b2):
    full = lambda i: (0, 0)
    return pl.pallas_call(
        _ln2_body,
        grid=(N // NBLK,),
        in_specs=[
            pl.BlockSpec((NBLK, HIDDEN + HEADS), lambda i: (i, 0)),
            pl.BlockSpec((HEADS, HIDDEN), full),
            pl.BlockSpec((NBLK, HIDDEN), lambda i: (i, 0)),
            pl.BlockSpec((1, HIDDEN), full),
            pl.BlockSpec((1, HIDDEN), full),
            pl.BlockSpec((1, HIDDEN), full),
            pl.BlockSpec((1, HIDDEN), full),
            pl.BlockSpec((1, HIDDEN), full),
        ],
        out_specs=pl.BlockSpec((NBLK, HIDDEN), lambda i: (i, 0)),
        out_shape=jax.ShapeDtypeStruct((N, HIDDEN), jnp.float32),
    )(s, rexp, xp, bias, g1, b1, g2, b2)


def _pool_body(x_ref, b_ref, sum_ref, cnt_ref, max_ref):
    i = pl.program_id(0)
    x = x_ref[...]
    bb = b_ref[...]
    gids = jax.lax.broadcasted_iota(jnp.int32, (1, NUM_GRAPHS), 1)
    onehot = (bb == gids).astype(jnp.float32)
    psum = jax.lax.dot_general(onehot, x, (((0,), (0,)), ((), ())),
                               preferred_element_type=jnp.float32, precision=jax.lax.Precision.HIGHEST)
    ones = jnp.ones((x.shape[0], 1), jnp.float32)
    pcnt = jax.lax.dot_general(onehot, ones, (((0,), (0,)), ((), ())),
                               preferred_element_type=jnp.float32, precision=jax.lax.Precision.HIGHEST)
    neg = jnp.float32(-3.0e38)
    rows = []
    for g in range(NUM_GRAPHS):
        mask = bb == g
        rows.append(jnp.max(jnp.where(mask, x, neg), axis=0, keepdims=True))
    pmax = jnp.concatenate(rows, axis=0)

    @pl.when(i == 0)
    def _():
        sum_ref[...] = psum
        cnt_ref[...] = pcnt
        max_ref[...] = pmax

    @pl.when(i > 0)
    def _():
        sum_ref[...] = sum_ref[...] + psum
        cnt_ref[...] = cnt_ref[...] + pcnt
        max_ref[...] = jnp.maximum(max_ref[...], pmax)


def _pool(x, batch2d):
    full = lambda i: (0, 0)
    return pl.pallas_call(
        _pool_body,
        grid=(N // NBLK,),
        in_specs=[
            pl.BlockSpec((NBLK, HIDDEN), lambda i: (i, 0)),
            pl.BlockSpec((NBLK, 1), lambda i: (i, 0)),
        ],
        out_specs=[
            pl.BlockSpec((NUM_GRAPHS, HIDDEN), full),
            pl.BlockSpec((NUM_GRAPHS, 1), full),
            pl.BlockSpec((NUM_GRAPHS, HIDDEN), full),
        ],
        out_shape=[
            jax.ShapeDtypeStruct((NUM_GRAPHS, HIDDEN), jnp.float32),
            jax.ShapeDtypeStruct((NUM_GRAPHS, 1), jnp.float32),
            jax.ShapeDtypeStruct((NUM_GRAPHS, HIDDEN), jnp.float32),
        ],
    )(x, batch2d)


def _head_body(xsum_ref, cnt_ref, xmax_ref, wpa_ref, wpb_ref, bp_ref,
               wc1_ref, bc1_ref, wc2_ref, bc2_ref,
               we1_ref, be1_ref, we2_ref, be2_ref,
               logits_ref, energy_ref):
    cnt = jnp.maximum(cnt_ref[...], 1.0)
    xmean = xsum_ref[...] / cnt
    xm = xmax_ref[...]
    xm = jnp.where(xm > jnp.float32(-1.0e38), xm, 0.0)
    pool = jnp.dot(xmean, wpa_ref[...], preferred_element_type=jnp.float32, precision=jax.lax.Precision.HIGHEST)
    pool = pool + jnp.dot(xm, wpb_ref[...], preferred_element_type=jnp.float32, precision=jax.lax.Precision.HIGHEST)
    pool = jnp.maximum(pool + bp_ref[...], 0.0)
    h1 = jnp.maximum(
        jnp.dot(pool, wc1_ref[...], preferred_element_type=jnp.float32, precision=jax.lax.Precision.HIGHEST) + bc1_ref[...], 0.0)
    logits_ref[...] = jnp.dot(h1, wc2_ref[...], preferred_element_type=jnp.float32, precision=jax.lax.Precision.HIGHEST) + bc2_ref[...]
    e1 = jnp.maximum(
        jnp.dot(pool, we1_ref[...], preferred_element_type=jnp.float32, precision=jax.lax.Precision.HIGHEST) + be1_ref[...], 0.0)
    z = jnp.dot(e1, we2_ref[...], preferred_element_type=jnp.float32, precision=jax.lax.Precision.HIGHEST) + be2_ref[...]
    energy_ref[...] = jnp.maximum(z, 0.0) + jnp.log(1.0 + jnp.exp(-jnp.abs(z)))


def _heads(xsum, cnt, xmax, p):
    wpa = p["pool"]["W"][:, :HIDDEN].T
    wpb = p["pool"]["W"][:, HIDDEN:].T
    bp = p["pool"]["b"].reshape(1, -1)
    wc1 = p["cls1"]["W"].T
    bc1 = p["cls1"]["b"].reshape(1, -1)
    wc2 = p["cls2"]["W"].T
    bc2 = p["cls2"]["b"].reshape(1, -1)
    we1 = p["en1"]["W"].T
    be1 = p["en1"]["b"].reshape(1, -1)
    we2 = p["en2"]["W"].T
    be2 = p["en2"]["b"].reshape(1, -1)
    return pl.pallas_call(
        _head_body,
        out_shape=[
            jax.ShapeDtypeStruct((NUM_GRAPHS, 5), jnp.float32),
            jax.ShapeDtypeStruct((NUM_GRAPHS, 1), jnp.float32),
        ],
    )(xsum, cnt, xmax, wpa, wpb, bp, wc1, bc1, wc2, bc2, we1, be1, we2, be2)


def kernel(x, edge_index, batch, params):
    row = edge_index[0]
    col = edge_index[1]

    encw = params["enc_lin"]["W"].T
    s0, t0 = _bn_fold(params["enc_bn"])
    t0 = t0 + params["enc_lin"]["b"].reshape(1, -1) * s0
    h = _encoder(x, encw, s0, t0)

    for i in range(NUM_BLOCKS):
        pc = params["edge_convs"][i]
        w1 = pc["lin1"]["W"]
        w1aT = w1[:, :HIDDEN].T
        w1bT = w1[:, HIDDEN:].T
        s1, t1 = _bn_fold(pc["bn1"])
        t1 = t1 + pc["lin1"]["b"].reshape(1, -1) * s1
        w2T = pc["lin2"]["W"].T
        s2, t2 = _bn_fold(pc["bn2"])
        t2 = t2 + pc["lin2"]["b"].reshape(1, -1) * s2

        xr, xc = _sc_gather_multi([(h, row), (h, col)])
        he = _edge_mlp(xr, xc, w1aT, w1bT, s1, t1, w2T, s2, t2)
        ec = jax.ops.segment_sum(he, row, num_segments=N)
        ln = params["lns"][2 * i]
        x1 = _res_ln(ec, h, ln["g"].reshape(1, -1), ln["b"].reshape(1, -1))

        g = params["gats"][i]
        idx_h = jnp.repeat(jnp.arange(HEADS), HEAD_DIM)
        ms = jnp.zeros((HEADS, HIDDEN), jnp.float32)
        ms = ms.at[idx_h, jnp.arange(HIDDEN)].set(g["att_src"].reshape(-1))
        md = jnp.zeros((HEADS, HIDDEN), jnp.float32)
        md = md.at[idx_h, jnp.arange(HIDDEN)].set(g["att_dst"].reshape(-1))
        mT = jnp.concatenate([ms, md], axis=0).T
        haa, aa, cm = _gat_proj(x1, g["W"].T, mT)
        c = cm[:, :HEADS] + cm[:, HEADS:]
        har, ac = _sc_gather_multi([(haa, row), (aa, col)])
        rexp = jnp.zeros((HEADS, HIDDEN), jnp.float32)
        rexp = rexp.at[idx_h, jnp.arange(HIDDEN)].set(1.0)
        y = _attn_msg(har, ac, c, rexp)
        s = jax.ops.segment_sum(y, col, num_segments=N)
        ln2 = params["lns"][2 * i + 1]
        h = _gat_epilogue(s, rexp, x1, g["bias"].reshape(1, -1),
                          g["ln_g"].reshape(1, -1), g["ln_b"].reshape(1, -1),
                          ln2["g"].reshape(1, -1), ln2["b"].reshape(1, -1))

    xsum, cnt, xmax = _pool(h, batch.reshape(-1, 1).astype(jnp.int32))
    logits, energy = _heads(xsum, cnt, xmax, params)
    return logits, energy


# speedup vs baseline: 18.7294x; 1.1951x over previous
import functools

import jax
import jax.numpy as jnp
import numpy as np
from jax import lax
from jax.experimental import pallas as pl
from jax.experimental.pallas import tpu as pltpu
from jax.experimental.pallas import tpu_sc as plsc

N = 50000
E = 800000
HIDDEN = 64
HEADS = 8
HEAD_DIM = 8
NUM_GRAPHS = 64
NUM_BLOCKS = 3

EBLK = 6400
NBLK = 5000


def _bn_fold(bn):
    s = bn["g"] / jnp.sqrt(bn["rv"] + 1e-5)
    t = bn["b"] - bn["rm"] * s
    return s.reshape(1, -1), t.reshape(1, -1)


def _ln_expr(z, g, b):
    m = jnp.mean(z, axis=-1, keepdims=True)
    v = jnp.mean((z - m) ** 2, axis=-1, keepdims=True)
    return (z - m) * jax.lax.rsqrt(v + 1e-5) * g + b


NWORKERS = 32
G_UNIT = 128
G_GROUP = 512
HALF_A = 409600
HALF_B = E - HALF_A


def _sc_gather_multi(pairs, estart, elen):
    dims = tuple(int(t.shape[1]) for t, _ in pairs)
    np_ = len(pairs)
    udims = sorted(set(dims))
    epw = elen // NWORKERS
    assert elen % NWORKERS == 0 and epw % 8 == 0 and estart % 8 == 0
    ngrp = epw // G_GROUP
    npair = ngrp // 2
    left = ngrp - 2 * npair
    rem = epw - ngrp * G_GROUP
    remu = rem // G_UNIT
    tail = rem - remu * G_UNIT

    scratch = [pltpu.VMEM((epw,), jnp.int32)]
    for d in udims:
        scratch += [pltpu.VMEM((G_GROUP, d), jnp.float32)] * 2
    scratch += [pltpu.SemaphoreType.DMA] * 4

    @functools.partial(
        pl.kernel,
        mesh=plsc.VectorSubcoreMesh(core_axis_name="c", subcore_axis_name="s"),
        out_type=[jax.ShapeDtypeStruct((elen, d), jnp.float32) for d in dims],
        scratch_types=scratch,
        compiler_params=pltpu.CompilerParams(use_tc_tiling_on_sc=False),
    )
    def k(*refs):
        tables = refs[:np_]
        idxs = refs[np_:2 * np_]
        outs = refs[2 * np_:3 * np_]
        idxb = refs[3 * np_]
        pos = 3 * np_ + 1
        bufmap = {}
        for d in udims:
            bufmap[d] = (refs[pos], refs[pos + 1])
            pos += 2
        gs = refs[pos:pos + 2]
        ws = refs[pos + 2:pos + 4]
        w = lax.axis_index("s") * 2 + lax.axis_index("c")
        base_i = estart + w * epw
        base_o = w * epw

        for p in range(np_):
            tab, idx, out = tables[p], idxs[p], outs[p]
            rb = bufmap[dims[p]]
            pltpu.sync_copy(idx.at[pl.ds(base_i, epw)], idxb)

            def grp_dma(grp, b, issue, nu=4):
                for k_ in range(nu):
                    off = grp * G_GROUP + k_ * G_UNIT
                    c = pltpu.make_async_copy(
                        tab.at[idxb.at[pl.ds(off, G_UNIT)]],
                        rb[b].at[pl.ds(k_ * G_UNIT, G_UNIT)], gs[b])
                    c.start() if issue else c.wait()

            def wb(grp, b, nrows=G_GROUP):
                return pltpu.make_async_copy(
                    rb[b].at[pl.ds(0, nrows)],
                    out.at[pl.ds(base_o + grp * G_GROUP, nrows)], ws[b])

            if npair > 0:
                def body(jj, carry):
                    for b in range(2):
                        grp = 2 * jj + b

                        @pl.when(jj > 0)
                        def _():
                            wb(grp - 2, b).wait()
                        grp_dma(grp, b, True)
                    for b in range(2):
                        grp = 2 * jj + b
                        grp_dma(grp, b, False)
                        wb(grp, b).start()
                    return carry

                lax.fori_loop(0, npair, body, 0)
                for b in range(2):
                    wb(2 * npair - 2 + b, b).wait()

            if left:
                grp = 2 * npair
                grp_dma(grp, 0, True)
                grp_dma(grp, 0, False)
                wb(grp, 0).start()
                wb(grp, 0).wait()

            if rem:
                rem_base = ngrp * G_GROUP
                for k_ in range(remu):
                    pltpu.async_copy(
                        tab.at[idxb.at[pl.ds(rem_base + k_ * G_UNIT, G_UNIT)]],
                        rb[0].at[pl.ds(k_ * G_UNIT, G_UNIT)], gs[0])
                if tail:
                    pltpu.async_copy(
                        tab.at[idxb.at[pl.ds(rem_base + remu * G_UNIT, tail)]],
                        rb[0].at[pl.ds(remu * G_UNIT, tail)], gs[0])
                for k_ in range(remu):
                    pltpu.make_async_copy(
                        tab.at[idxb.at[pl.ds(rem_base + k_ * G_UNIT, G_UNIT)]],
                        rb[0].at[pl.ds(k_ * G_UNIT, G_UNIT)], gs[0]).wait()
                if tail:
                    pltpu.make_async_copy(
                        tab.at[idxb.at[pl.ds(rem_base + remu * G_UNIT, tail)]],
                        rb[0].at[pl.ds(remu * G_UNIT, tail)], gs[0]).wait()
                wb(ngrp, 0, rem).start()
                wb(ngrp, 0, rem).wait()

    res = k(*[t for t, _ in pairs], *[i for _, i in pairs])
    return list(res) if np_ > 1 else [res]


def _enc_body(x_ref, w_ref, s_ref, t_ref, o_ref):
    y = jnp.dot(x_ref[...], w_ref[...], preferred_element_type=jnp.float32, precision=jax.lax.Precision.HIGHEST)
    o_ref[...] = jnp.maximum(y * s_ref[...] + t_ref[...], 0.0)


def _encoder(x, wT, s, t):
    return pl.pallas_call(
        _enc_body,
        grid=(N // NBLK,),
        in_specs=[
            pl.BlockSpec((NBLK, 4), lambda i: (i, 0)),
            pl.BlockSpec((4, HIDDEN), lambda i: (0, 0)),
            pl.BlockSpec((1, HIDDEN), lambda i: (0, 0)),
            pl.BlockSpec((1, HIDDEN), lambda i: (0, 0)),
        ],
        out_specs=pl.BlockSpec((NBLK, HIDDEN), lambda i: (i, 0)),
        out_shape=jax.ShapeDtypeStruct((N, HIDDEN), jnp.float32),
    )(x, wT, s, t)


def _edge_mlp_body(xr_ref, xc_ref, w1a_ref, w1b_ref, s1_ref, t1_ref,
                   w2_ref, s2_ref, t2_ref, o_ref):
    acc = jnp.dot(xr_ref[...], w1a_ref[...], preferred_element_type=jnp.float32, precision=jax.lax.Precision.HIGHEST)
    acc = acc + jnp.dot(xc_ref[...], w1b_ref[...], preferred_element_type=jnp.float32, precision=jax.lax.Precision.HIGHEST)
    h = jnp.maximum(acc * s1_ref[...] + t1_ref[...], 0.0)
    h2 = jnp.dot(h, w2_ref[...], preferred_element_type=jnp.float32, precision=jax.lax.Precision.HIGHEST)
    o_ref[...] = jnp.maximum(h2 * s2_ref[...] + t2_ref[...], 0.0)


def _edge_mlp(xr, xc, w1aT, w1bT, s1, t1, w2T, s2, t2):
    full = lambda i: (0, 0)
    return pl.pallas_call(
        _edge_mlp_body,
        grid=(xr.shape[0] // EBLK,),
        in_specs=[
            pl.BlockSpec((EBLK, HIDDEN), lambda i: (i, 0)),
            pl.BlockSpec((EBLK, HIDDEN), lambda i: (i, 0)),
            pl.BlockSpec((HIDDEN, HIDDEN), full),
            pl.BlockSpec((HIDDEN, HIDDEN), full),
            pl.BlockSpec((1, HIDDEN), full),
            pl.BlockSpec((1, HIDDEN), full),
            pl.BlockSpec((HIDDEN, HIDDEN), full),
            pl.BlockSpec((1, HIDDEN), full),
            pl.BlockSpec((1, HIDDEN), full),
        ],
        out_specs=pl.BlockSpec((EBLK, HIDDEN), lambda i: (i, 0)),
        out_shape=jax.ShapeDtypeStruct((xr.shape[0], HIDDEN), jnp.float32),
    )(xr, xc, w1aT, w1bT, s1, t1, w2T, s2, t2)


def _ln1_body(y0_ref, y1_ref, r_ref, g_ref, b_ref, o_ref):
    z = y0_ref[...] + y1_ref[...] + r_ref[...]
    o_ref[...] = _ln_expr(z, g_ref[...], b_ref[...])


def _res_ln(y0, y1, r, g, b):
    full = lambda i: (0, 0)
    return pl.pallas_call(
        _ln1_body,
        grid=(N // NBLK,),
        in_specs=[
            pl.BlockSpec((NBLK, HIDDEN), lambda i: (i, 0)),
            pl.BlockSpec((NBLK, HIDDEN), lambda i: (i, 0)),
            pl.BlockSpec((NBLK, HIDDEN), lambda i: (i, 0)),
            pl.BlockSpec((1, HIDDEN), full),
            pl.BlockSpec((1, HIDDEN), full),
        ],
        out_specs=pl.BlockSpec((NBLK, HIDDEN), lambda i: (i, 0)),
        out_shape=jax.ShapeDtypeStruct((N, HIDDEN), jnp.float32),
    )(y0, y1, r, g, b)


def _gatproj_body(x_ref, w_ref, m_ref, haa_ref, aa_ref, cm_ref):
    i = pl.program_id(0)
    h = jnp.dot(x_ref[...], w_ref[...], preferred_element_type=jnp.float32, precision=jax.lax.Precision.HIGHEST)
    haa_ref[:, :HIDDEN] = h
    aa = jnp.dot(h, m_ref[...], preferred_element_type=jnp.float32, precision=jax.lax.Precision.HIGHEST)
    haa_ref[:, HIDDEN:] = aa
    aa_ref[...] = aa
    bm = jnp.max(aa, axis=0, keepdims=True)

    @pl.when(i == 0)
    def _():
        cm_ref[...] = bm

    @pl.when(i > 0)
    def _():
        cm_ref[...] = jnp.maximum(cm_ref[...], bm)


def _gat_proj(x, wT, mT):
    full = lambda i: (0, 0)
    return pl.pallas_call(
        _gatproj_body,
        grid=(N // NBLK,),
        in_specs=[
            pl.BlockSpec((NBLK, HIDDEN), lambda i: (i, 0)),
            pl.BlockSpec((HIDDEN, HIDDEN), full),
            pl.BlockSpec((HIDDEN, 2 * HEADS), full),
        ],
        out_specs=[
            pl.BlockSpec((NBLK, HIDDEN + 2 * HEADS), lambda i: (i, 0)),
            pl.BlockSpec((NBLK, 2 * HEADS), lambda i: (i, 0)),
            pl.BlockSpec((1, 2 * HEADS), full),
        ],
        out_shape=[
            jax.ShapeDtypeStruct((N, HIDDEN + 2 * HEADS), jnp.float32),
            jax.ShapeDtypeStruct((N, 2 * HEADS), jnp.float32),
            jax.ShapeDtypeStruct((1, 2 * HEADS), jnp.float32),
        ],
    )(x, wT, mT)


def _attnmsg_body(har_ref, ac_ref, c_ref, r_ref, y_ref):
    z = har_ref[:, HIDDEN:HIDDEN + HEADS] + ac_ref[:, HEADS:]
    a = jnp.where(z >= 0.0, z, 0.2 * z)
    ex = jnp.exp(a - c_ref[...])
    ex64 = jnp.dot(ex, r_ref[...], preferred_element_type=jnp.float32, precision=jax.lax.Precision.HIGHEST)
    y_ref[:, :HIDDEN] = har_ref[:, :HIDDEN] * ex64
    y_ref[:, HIDDEN:] = ex


def _attn_msg(har, ac, c, rexp):
    full = lambda i: (0, 0)
    return pl.pallas_call(
        _attnmsg_body,
        grid=(har.shape[0] // EBLK,),
        in_specs=[
            pl.BlockSpec((EBLK, HIDDEN + 2 * HEADS), lambda i: (i, 0)),
            pl.BlockSpec((EBLK, 2 * HEADS), lambda i: (i, 0)),
            pl.BlockSpec((1, HEADS), full),
            pl.BlockSpec((HEADS, HIDDEN), full),
        ],
        out_specs=pl.BlockSpec((EBLK, HIDDEN + HEADS), lambda i: (i, 0)),
        out_shape=jax.ShapeDtypeStruct((har.shape[0], HIDDEN + HEADS), jnp.float32),
    )(har, ac, c, rexp)


def _ln2_body(s0_ref, s1_ref, r_ref, xp_ref, bias_ref,
              g1_ref, b1_ref, g2_ref, b2_ref, o_ref):
    s_ = s0_ref[...] + s1_ref[...]
    dd = jnp.dot(s_[:, HIDDEN:], r_ref[...], preferred_element_type=jnp.float32, precision=jax.lax.Precision.HIGHEST)
    xg = s_[:, :HIDDEN] / (dd + 1e-16)
    t = _ln_expr(xg + bias_ref[...] + xp_ref[...], g1_ref[...], b1_ref[...])
    o_ref[...] = _ln_expr(t + xp_ref[...], g2_ref[...], b2_ref[...])


def _gat_epilogue(s0, s1, rexp, xp, bias, g1, b1, g2, b2):
    full = lambda i: (0, 0)
    return pl.pallas_call(
        _ln2_body,
        grid=(N // NBLK,),
        in_specs=[
            pl.BlockSpec((NBLK, HIDDEN + HEADS), lambda i: (i, 0)),
            pl.BlockSpec((NBLK, HIDDEN + HEADS), lambda i: (i, 0)),
            pl.BlockSpec((HEADS, HIDDEN), full),
            pl.BlockSpec((NBLK, HIDDEN), lambda i: (i, 0)),
            pl.BlockSpec((1, HIDDEN), full),
            pl.BlockSpec((1, HIDDEN), full),
            pl.BlockSpec((1, HIDDEN), full),
            pl.BlockSpec((1, HIDDEN), full),
            pl.BlockSpec((1, HIDDEN), full),
        ],
        out_specs=pl.BlockSpec((NBLK, HIDDEN), lambda i: (i, 0)),
        out_shape=jax.ShapeDtypeStruct((N, HIDDEN), jnp.float32),
    )(s0, s1, rexp, xp, bias, g1, b1, g2, b2)


def _pool_body(x_ref, b_ref, sum_ref, cnt_ref, max_ref):
    i = pl.program_id(0)
    x = x_ref[...]
    bb = b_ref[...]
    gids = jax.lax.broadcasted_iota(jnp.int32, (1, NUM_GRAPHS), 1)
    onehot = (bb == gids).astype(jnp.float32)
    psum = jax.lax.dot_general(onehot, x, (((0,), (0,)), ((), ())),
                               preferred_element_type=jnp.float32, precision=jax.lax.Precision.HIGHEST)
    ones = jnp.ones((x.shape[0], 1), jnp.float32)
    pcnt = jax.lax.dot_general(onehot, ones, (((0,), (0,)), ((), ())),
                               preferred_element_type=jnp.float32, precision=jax.lax.Precision.HIGHEST)
    neg = jnp.float32(-3.0e38)
    rows = []
    for g in range(NUM_GRAPHS):
        mask = bb == g
        rows.append(jnp.max(jnp.where(mask, x, neg), axis=0, keepdims=True))
    pmax = jnp.concatenate(rows, axis=0)

    @pl.when(i == 0)
    def _():
        sum_ref[...] = psum
        cnt_ref[...] = pcnt
        max_ref[...] = pmax

    @pl.when(i > 0)
    def _():
        sum_ref[...] = sum_ref[...] + psum
        cnt_ref[...] = cnt_ref[...] + pcnt
        max_ref[...] = jnp.maximum(max_ref[...], pmax)


def _pool(x, batch2d):
    full = lambda i: (0, 0)
    return pl.pallas_call(
        _pool_body,
        grid=(N // NBLK,),
        in_specs=[
            pl.BlockSpec((NBLK, HIDDEN), lambda i: (i, 0)),
            pl.BlockSpec((NBLK, 1), lambda i: (i, 0)),
        ],
        out_specs=[
            pl.BlockSpec((NUM_GRAPHS, HIDDEN), full),
            pl.BlockSpec((NUM_GRAPHS, 1), full),
            pl.BlockSpec((NUM_GRAPHS, HIDDEN), full),
        ],
        out_shape=[
            jax.ShapeDtypeStruct((NUM_GRAPHS, HIDDEN), jnp.float32),
            jax.ShapeDtypeStruct((NUM_GRAPHS, 1), jnp.float32),
            jax.ShapeDtypeStruct((NUM_GRAPHS, HIDDEN), jnp.float32),
        ],
    )(x, batch2d)


def _head_body(xsum_ref, cnt_ref, xmax_ref, wpa_ref, wpb_ref, bp_ref,
               wc1_ref, bc1_ref, wc2_ref, bc2_ref,
               we1_ref, be1_ref, we2_ref, be2_ref,
               logits_ref, energy_ref):
    cnt = jnp.maximum(cnt_ref[...], 1.0)
    xmean = xsum_ref[...] / cnt
    xm = xmax_ref[...]
    xm = jnp.where(xm > jnp.float32(-1.0e38), xm, 0.0)
    pool = jnp.dot(xmean, wpa_ref[...], preferred_element_type=jnp.float32, precision=jax.lax.Precision.HIGHEST)
    pool = pool + jnp.dot(xm, wpb_ref[...], preferred_element_type=jnp.float32, precision=jax.lax.Precision.HIGHEST)
    pool = jnp.maximum(pool + bp_ref[...], 0.0)
    h1 = jnp.maximum(
        jnp.dot(pool, wc1_ref[...], preferred_element_type=jnp.float32, precision=jax.lax.Precision.HIGHEST) + bc1_ref[...], 0.0)
    logits_ref[...] = jnp.dot(h1, wc2_ref[...], preferred_element_type=jnp.float32, precision=jax.lax.Precision.HIGHEST) + bc2_ref[...]
    e1 = jnp.maximum(
        jnp.dot(pool, we1_ref[...], preferred_element_type=jnp.float32, precision=jax.lax.Precision.HIGHEST) + be1_ref[...], 0.0)
    z = jnp.dot(e1, we2_ref[...], preferred_element_type=jnp.float32, precision=jax.lax.Precision.HIGHEST) + be2_ref[...]
    energy_ref[...] = jnp.maximum(z, 0.0) + jnp.log(1.0 + jnp.exp(-jnp.abs(z)))


def _heads(xsum, cnt, xmax, p):
    wpa = p["pool"]["W"][:, :HIDDEN].T
    wpb = p["pool"]["W"][:, HIDDEN:].T
    bp = p["pool"]["b"].reshape(1, -1)
    wc1 = p["cls1"]["W"].T
    bc1 = p["cls1"]["b"].reshape(1, -1)
    wc2 = p["cls2"]["W"].T
    bc2 = p["cls2"]["b"].reshape(1, -1)
    we1 = p["en1"]["W"].T
    be1 = p["en1"]["b"].reshape(1, -1)
    we2 = p["en2"]["W"].T
    be2 = p["en2"]["b"].reshape(1, -1)
    return pl.pallas_call(
        _head_body,
        out_shape=[
            jax.ShapeDtypeStruct((NUM_GRAPHS, 5), jnp.float32),
            jax.ShapeDtypeStruct((NUM_GRAPHS, 1), jnp.float32),
        ],
    )(xsum, cnt, xmax, wpa, wpb, bp, wc1, bc1, wc2, bc2, we1, be1, we2, be2)


def kernel(x, edge_index, batch, params):
    row = edge_index[0]
    col = edge_index[1]

    encw = params["enc_lin"]["W"].T
    s0, t0 = _bn_fold(params["enc_bn"])
    t0 = t0 + params["enc_lin"]["b"].reshape(1, -1) * s0
    h = _encoder(x, encw, s0, t0)

    for i in range(NUM_BLOCKS):
        pc = params["edge_convs"][i]
        w1 = pc["lin1"]["W"]
        w1aT = w1[:, :HIDDEN].T
        w1bT = w1[:, HIDDEN:].T
        s1, t1 = _bn_fold(pc["bn1"])
        t1 = t1 + pc["lin1"]["b"].reshape(1, -1) * s1
        w2T = pc["lin2"]["W"].T
        s2, t2 = _bn_fold(pc["bn2"])
        t2 = t2 + pc["lin2"]["b"].reshape(1, -1) * s2

        ec_parts = []
        for es, el in ((0, HALF_A), (HALF_A, HALF_B)):
            xr, xc = _sc_gather_multi([(h, row), (h, col)], es, el)
            he = _edge_mlp(xr, xc, w1aT, w1bT, s1, t1, w2T, s2, t2)
            ec_parts.append(jax.ops.segment_sum(he, row[es:es + el], num_segments=N))
        ln = params["lns"][2 * i]
        x1 = _res_ln(ec_parts[0], ec_parts[1], h,
                     ln["g"].reshape(1, -1), ln["b"].reshape(1, -1))

        g = params["gats"][i]
        idx_h = jnp.repeat(jnp.arange(HEADS), HEAD_DIM)
        ms = jnp.zeros((HEADS, HIDDEN), jnp.float32)
        ms = ms.at[idx_h, jnp.arange(HIDDEN)].set(g["att_src"].reshape(-1))
        md = jnp.zeros((HEADS, HIDDEN), jnp.float32)
        md = md.at[idx_h, jnp.arange(HIDDEN)].set(g["att_dst"].reshape(-1))
        mT = jnp.concatenate([ms, md], axis=0).T
        haa, aa, cm = _gat_proj(x1, g["W"].T, mT)
        c = cm[:, :HEADS] + cm[:, HEADS:]
        rexp = jnp.zeros((HEADS, HIDDEN), jnp.float32)
        rexp = rexp.at[idx_h, jnp.arange(HIDDEN)].set(1.0)
        s_parts = []
        for es, el in ((0, HALF_A), (HALF_A, HALF_B)):
            har, ac = _sc_gather_multi([(haa, row), (aa, col)], es, el)
            y = _attn_msg(har, ac, c, rexp)
            s_parts.append(jax.ops.segment_sum(y, col[es:es + el], num_segments=N))
        ln2 = params["lns"][2 * i + 1]
        h = _gat_epilogue(s_parts[0], s_parts[1], rexp, x1, g["bias"].reshape(1, -1),
                          g["ln_g"].reshape(1, -1), g["ln_b"].reshape(1, -1),
                          ln2["g"].reshape(1, -1), ln2["b"].reshape(1, -1))

    xsum, cnt, xmax = _pool(h, batch.reshape(-1, 1).astype(jnp.int32))
    logits, energy = _heads(xsum, cnt, xmax, params)
    return logits, energy


# speedup vs baseline: 19.8189x; 1.0582x over previous
import functools

import jax
import jax.numpy as jnp
import numpy as np
from jax import lax
from jax.experimental import pallas as pl
from jax.experimental.pallas import tpu as pltpu
from jax.experimental.pallas import tpu_sc as plsc

N = 50000
E = 800000
HIDDEN = 64
HEADS = 8
HEAD_DIM = 8
NUM_GRAPHS = 64
NUM_BLOCKS = 3

EBLK = 6400
NBLK = 5000


def _bn_fold(bn):
    s = bn["g"] / jnp.sqrt(bn["rv"] + 1e-5)
    t = bn["b"] - bn["rm"] * s
    return s.reshape(1, -1), t.reshape(1, -1)


def _ln_expr(z, g, b):
    m = jnp.mean(z, axis=-1, keepdims=True)
    v = jnp.mean((z - m) ** 2, axis=-1, keepdims=True)
    return (z - m) * jax.lax.rsqrt(v + 1e-5) * g + b


NWORKERS = 32
G_UNIT = 128
G_GROUP = 512
ERANGES = ((0, 204800), (204800, 198400), (403200, 198400), (601600, 198400))


def _sc_gather_multi(pairs, estart, elen):
    dims = tuple(int(t.shape[1]) for t, _ in pairs)
    np_ = len(pairs)
    udims = sorted(set(dims))
    epw = elen // NWORKERS
    assert elen % NWORKERS == 0 and epw % 8 == 0 and estart % 8 == 0
    ngrp = epw // G_GROUP
    npair = ngrp // 2
    left = ngrp - 2 * npair
    rem = epw - ngrp * G_GROUP
    remu = rem // G_UNIT
    tail = rem - remu * G_UNIT

    scratch = [pltpu.VMEM((epw,), jnp.int32)]
    for d in udims:
        scratch += [pltpu.VMEM((G_GROUP, d), jnp.float32)] * 2
    scratch += [pltpu.SemaphoreType.DMA] * 4

    @functools.partial(
        pl.kernel,
        mesh=plsc.VectorSubcoreMesh(core_axis_name="c", subcore_axis_name="s"),
        out_type=[jax.ShapeDtypeStruct((elen, d), jnp.float32) for d in dims],
        scratch_types=scratch,
        compiler_params=pltpu.CompilerParams(use_tc_tiling_on_sc=False),
    )
    def k(*refs):
        tables = refs[:np_]
        idxs = refs[np_:2 * np_]
        outs = refs[2 * np_:3 * np_]
        idxb = refs[3 * np_]
        pos = 3 * np_ + 1
        bufmap = {}
        for d in udims:
            bufmap[d] = (refs[pos], refs[pos + 1])
            pos += 2
        gs = refs[pos:pos + 2]
        ws = refs[pos + 2:pos + 4]
        w = lax.axis_index("s") * 2 + lax.axis_index("c")
        base_i = estart + w * epw
        base_o = w * epw

        for p in range(np_):
            tab, idx, out = tables[p], idxs[p], outs[p]
            rb = bufmap[dims[p]]
            pltpu.sync_copy(idx.at[pl.ds(base_i, epw)], idxb)

            def grp_dma(grp, b, issue, nu=4):
                for k_ in range(nu):
                    off = grp * G_GROUP + k_ * G_UNIT
                    c = pltpu.make_async_copy(
                        tab.at[idxb.at[pl.ds(off, G_UNIT)]],
                        rb[b].at[pl.ds(k_ * G_UNIT, G_UNIT)], gs[b])
                    c.start() if issue else c.wait()

            def wb(grp, b, nrows=G_GROUP):
                return pltpu.make_async_copy(
                    rb[b].at[pl.ds(0, nrows)],
                    out.at[pl.ds(base_o + grp * G_GROUP, nrows)], ws[b])

            if npair > 0:
                def body(jj, carry):
                    for b in range(2):
                        grp = 2 * jj + b

                        @pl.when(jj > 0)
                        def _():
                            wb(grp - 2, b).wait()
                        grp_dma(grp, b, True)
                    for b in range(2):
                        grp = 2 * jj + b
                        grp_dma(grp, b, False)
                        wb(grp, b).start()
                    return carry

                lax.fori_loop(0, npair, body, 0)
                for b in range(2):
                    wb(2 * npair - 2 + b, b).wait()

            if left:
                grp = 2 * npair
                grp_dma(grp, 0, True)
                grp_dma(grp, 0, False)
                wb(grp, 0).start()
                wb(grp, 0).wait()

            if rem:
                rem_base = ngrp * G_GROUP
                for k_ in range(remu):
                    pltpu.async_copy(
                        tab.at[idxb.at[pl.ds(rem_base + k_ * G_UNIT, G_UNIT)]],
                        rb[0].at[pl.ds(k_ * G_UNIT, G_UNIT)], gs[0])
                if tail:
                    pltpu.async_copy(
                        tab.at[idxb.at[pl.ds(rem_base + remu * G_UNIT, tail)]],
                        rb[0].at[pl.ds(remu * G_UNIT, tail)], gs[0])
                for k_ in range(remu):
                    pltpu.make_async_copy(
                        tab.at[idxb.at[pl.ds(rem_base + k_ * G_UNIT, G_UNIT)]],
                        rb[0].at[pl.ds(k_ * G_UNIT, G_UNIT)], gs[0]).wait()
                if tail:
                    pltpu.make_async_copy(
                        tab.at[idxb.at[pl.ds(rem_base + remu * G_UNIT, tail)]],
                        rb[0].at[pl.ds(remu * G_UNIT, tail)], gs[0]).wait()
                wb(ngrp, 0, rem).start()
                wb(ngrp, 0, rem).wait()

    res = k(*[t for t, _ in pairs], *[i for _, i in pairs])
    return list(res) if np_ > 1 else [res]


def _enc_body(x_ref, w_ref, s_ref, t_ref, o_ref):
    y = jnp.dot(x_ref[...], w_ref[...], preferred_element_type=jnp.float32, precision=jax.lax.Precision.HIGHEST)
    o_ref[...] = jnp.maximum(y * s_ref[...] + t_ref[...], 0.0)


def _encoder(x, wT, s, t):
    return pl.pallas_call(
        _enc_body,
        grid=(N // NBLK,),
        in_specs=[
            pl.BlockSpec((NBLK, 4), lambda i: (i, 0)),
            pl.BlockSpec((4, HIDDEN), lambda i: (0, 0)),
            pl.BlockSpec((1, HIDDEN), lambda i: (0, 0)),
            pl.BlockSpec((1, HIDDEN), lambda i: (0, 0)),
        ],
        out_specs=pl.BlockSpec((NBLK, HIDDEN), lambda i: (i, 0)),
        out_shape=jax.ShapeDtypeStruct((N, HIDDEN), jnp.float32),
    )(x, wT, s, t)


def _edge_mlp_body(xr_ref, xc_ref, w1a_ref, w1b_ref, s1_ref, t1_ref,
                   w2_ref, s2_ref, t2_ref, o_ref):
    acc = jnp.dot(xr_ref[...], w1a_ref[...], preferred_element_type=jnp.float32, precision=jax.lax.Precision.HIGHEST)
    acc = acc + jnp.dot(xc_ref[...], w1b_ref[...], preferred_element_type=jnp.float32, precision=jax.lax.Precision.HIGHEST)
    h = jnp.maximum(acc * s1_ref[...] + t1_ref[...], 0.0)
    h2 = jnp.dot(h, w2_ref[...], preferred_element_type=jnp.float32, precision=jax.lax.Precision.HIGHEST)
    o_ref[...] = jnp.maximum(h2 * s2_ref[...] + t2_ref[...], 0.0)


def _edge_mlp(xr, xc, w1aT, w1bT, s1, t1, w2T, s2, t2):
    full = lambda i: (0, 0)
    return pl.pallas_call(
        _edge_mlp_body,
        grid=(xr.shape[0] // EBLK,),
        in_specs=[
            pl.BlockSpec((EBLK, HIDDEN), lambda i: (i, 0)),
            pl.BlockSpec((EBLK, HIDDEN), lambda i: (i, 0)),
            pl.BlockSpec((HIDDEN, HIDDEN), full),
            pl.BlockSpec((HIDDEN, HIDDEN), full),
            pl.BlockSpec((1, HIDDEN), full),
            pl.BlockSpec((1, HIDDEN), full),
            pl.BlockSpec((HIDDEN, HIDDEN), full),
            pl.BlockSpec((1, HIDDEN), full),
            pl.BlockSpec((1, HIDDEN), full),
        ],
        out_specs=pl.BlockSpec((EBLK, HIDDEN), lambda i: (i, 0)),
        out_shape=jax.ShapeDtypeStruct((xr.shape[0], HIDDEN), jnp.float32),
    )(xr, xc, w1aT, w1bT, s1, t1, w2T, s2, t2)


def _ln1_body(*refs):
    ys = refs[:len(ERANGES)]
    r_ref, g_ref, b_ref, o_ref = refs[len(ERANGES):]
    z = r_ref[...]
    for y in ys:
        z = z + y[...]
    o_ref[...] = _ln_expr(z, g_ref[...], b_ref[...])


def _res_ln(yparts, r, g, b):
    full = lambda i: (0, 0)
    blk = lambda i: (i, 0)
    return pl.pallas_call(
        _ln1_body,
        grid=(N // NBLK,),
        in_specs=[pl.BlockSpec((NBLK, HIDDEN), blk)] * (len(ERANGES) + 1) + [
            pl.BlockSpec((1, HIDDEN), full),
            pl.BlockSpec((1, HIDDEN), full),
        ],
        out_specs=pl.BlockSpec((NBLK, HIDDEN), blk),
        out_shape=jax.ShapeDtypeStruct((N, HIDDEN), jnp.float32),
    )(*yparts, r, g, b)


def _gatproj_body(x_ref, w_ref, m_ref, haa_ref, aa_ref, cm_ref):
    i = pl.program_id(0)
    h = jnp.dot(x_ref[...], w_ref[...], preferred_element_type=jnp.float32, precision=jax.lax.Precision.HIGHEST)
    haa_ref[:, :HIDDEN] = h
    aa = jnp.dot(h, m_ref[...], preferred_element_type=jnp.float32, precision=jax.lax.Precision.HIGHEST)
    haa_ref[:, HIDDEN:] = aa
    aa_ref[...] = aa
    bm = jnp.max(aa, axis=0, keepdims=True)

    @pl.when(i == 0)
    def _():
        cm_ref[...] = bm

    @pl.when(i > 0)
    def _():
        cm_ref[...] = jnp.maximum(cm_ref[...], bm)


def _gat_proj(x, wT, mT):
    full = lambda i: (0, 0)
    return pl.pallas_call(
        _gatproj_body,
        grid=(N // NBLK,),
        in_specs=[
            pl.BlockSpec((NBLK, HIDDEN), lambda i: (i, 0)),
            pl.BlockSpec((HIDDEN, HIDDEN), full),
            pl.BlockSpec((HIDDEN, 2 * HEADS), full),
        ],
        out_specs=[
            pl.BlockSpec((NBLK, HIDDEN + 2 * HEADS), lambda i: (i, 0)),
            pl.BlockSpec((NBLK, 2 * HEADS), lambda i: (i, 0)),
            pl.BlockSpec((1, 2 * HEADS), full),
        ],
        out_shape=[
            jax.ShapeDtypeStruct((N, HIDDEN + 2 * HEADS), jnp.float32),
            jax.ShapeDtypeStruct((N, 2 * HEADS), jnp.float32),
            jax.ShapeDtypeStruct((1, 2 * HEADS), jnp.float32),
        ],
    )(x, wT, mT)


def _attnmsg_body(har_ref, ac_ref, c_ref, r_ref, y_ref):
    z = har_ref[:, HIDDEN:HIDDEN + HEADS] + ac_ref[:, HEADS:]
    a = jnp.where(z >= 0.0, z, 0.2 * z)
    ex = jnp.exp(a - c_ref[...])
    ex64 = jnp.dot(ex, r_ref[...], preferred_element_type=jnp.float32, precision=jax.lax.Precision.HIGHEST)
    y_ref[:, :HIDDEN] = har_ref[:, :HIDDEN] * ex64
    y_ref[:, HIDDEN:] = ex


def _attn_msg(har, ac, c, rexp):
    full = lambda i: (0, 0)
    return pl.pallas_call(
        _attnmsg_body,
        grid=(har.shape[0] // EBLK,),
        in_specs=[
            pl.BlockSpec((EBLK, HIDDEN + 2 * HEADS), lambda i: (i, 0)),
            pl.BlockSpec((EBLK, 2 * HEADS), lambda i: (i, 0)),
            pl.BlockSpec((1, HEADS), full),
            pl.BlockSpec((HEADS, HIDDEN), full),
        ],
        out_specs=pl.BlockSpec((EBLK, HIDDEN + HEADS), lambda i: (i, 0)),
        out_shape=jax.ShapeDtypeStruct((har.shape[0], HIDDEN + HEADS), jnp.float32),
    )(har, ac, c, rexp)


def _ln2_body(*refs):
    ss = refs[:len(ERANGES)]
    r_ref, xp_ref, bias_ref, g1_ref, b1_ref, g2_ref, b2_ref, o_ref = refs[len(ERANGES):]
    s_ = ss[0][...]
    for sp in ss[1:]:
        s_ = s_ + sp[...]
    dd = jnp.dot(s_[:, HIDDEN:], r_ref[...], preferred_element_type=jnp.float32, precision=jax.lax.Precision.HIGHEST)
    xg = s_[:, :HIDDEN] / (dd + 1e-16)
    t = _ln_expr(xg + bias_ref[...] + xp_ref[...], g1_ref[...], b1_ref[...])
    o_ref[...] = _ln_expr(t + xp_ref[...], g2_ref[...], b2_ref[...])


def _gat_epilogue(sparts, rexp, xp, bias, g1, b1, g2, b2):
    full = lambda i: (0, 0)
    return pl.pallas_call(
        _ln2_body,
        grid=(N // NBLK,),
        in_specs=[
            pl.BlockSpec((NBLK, HIDDEN + HEADS), lambda i: (i, 0))] * len(ERANGES) + [
            pl.BlockSpec((HEADS, HIDDEN), full),
            pl.BlockSpec((NBLK, HIDDEN), lambda i: (i, 0)),
            pl.BlockSpec((1, HIDDEN), full),
            pl.BlockSpec((1, HIDDEN), full),
            pl.BlockSpec((1, HIDDEN), full),
            pl.BlockSpec((1, HIDDEN), full),
            pl.BlockSpec((1, HIDDEN), full),
        ],
        out_specs=pl.BlockSpec((NBLK, HIDDEN), lambda i: (i, 0)),
        out_shape=jax.ShapeDtypeStruct((N, HIDDEN), jnp.float32),
    )(*sparts, rexp, xp, bias, g1, b1, g2, b2)


def _pool_body(x_ref, b_ref, sum_ref, cnt_ref, max_ref):
    i = pl.program_id(0)
    x = x_ref[...]
    bb = b_ref[...]
    gids = jax.lax.broadcasted_iota(jnp.int32, (1, NUM_GRAPHS), 1)
    onehot = (bb == gids).astype(jnp.float32)
    psum = jax.lax.dot_general(onehot, x, (((0,), (0,)), ((), ())),
                               preferred_element_type=jnp.float32, precision=jax.lax.Precision.HIGHEST)
    ones = jnp.ones((x.shape[0], 1), jnp.float32)
    pcnt = jax.lax.dot_general(onehot, ones, (((0,), (0,)), ((), ())),
                               preferred_element_type=jnp.float32, precision=jax.lax.Precision.HIGHEST)
    neg = jnp.float32(-3.0e38)
    rows = []
    for g in range(NUM_GRAPHS):
        mask = bb == g
        rows.append(jnp.max(jnp.where(mask, x, neg), axis=0, keepdims=True))
    pmax = jnp.concatenate(rows, axis=0)

    @pl.when(i == 0)
    def _():
        sum_ref[...] = psum
        cnt_ref[...] = pcnt
        max_ref[...] = pmax

    @pl.when(i > 0)
    def _():
        sum_ref[...] = sum_ref[...] + psum
        cnt_ref[...] = cnt_ref[...] + pcnt
        max_ref[...] = jnp.maximum(max_ref[...], pmax)


def _pool(x, batch2d):
    full = lambda i: (0, 0)
    return pl.pallas_call(
        _pool_body,
        grid=(N // NBLK,),
        in_specs=[
            pl.BlockSpec((NBLK, HIDDEN), lambda i: (i, 0)),
            pl.BlockSpec((NBLK, 1), lambda i: (i, 0)),
        ],
        out_specs=[
            pl.BlockSpec((NUM_GRAPHS, HIDDEN), full),
            pl.BlockSpec((NUM_GRAPHS, 1), full),
            pl.BlockSpec((NUM_GRAPHS, HIDDEN), full),
        ],
        out_shape=[
            jax.ShapeDtypeStruct((NUM_GRAPHS, HIDDEN), jnp.float32),
            jax.ShapeDtypeStruct((NUM_GRAPHS, 1), jnp.float32),
            jax.ShapeDtypeStruct((NUM_GRAPHS, HIDDEN), jnp.float32),
        ],
    )(x, batch2d)


def _head_body(xsum_ref, cnt_ref, xmax_ref, wpa_ref, wpb_ref, bp_ref,
               wc1_ref, bc1_ref, wc2_ref, bc2_ref,
               we1_ref, be1_ref, we2_ref, be2_ref,
               logits_ref, energy_ref):
    cnt = jnp.maximum(cnt_ref[...], 1.0)
    xmean = xsum_ref[...] / cnt
    xm = xmax_ref[...]
    xm = jnp.where(xm > jnp.float32(-1.0e38), xm, 0.0)
    pool = jnp.dot(xmean, wpa_ref[...], preferred_element_type=jnp.float32, precision=jax.lax.Precision.HIGHEST)
    pool = pool + jnp.dot(xm, wpb_ref[...], preferred_element_type=jnp.float32, precision=jax.lax.Precision.HIGHEST)
    pool = jnp.maximum(pool + bp_ref[...], 0.0)
    h1 = jnp.maximum(
        jnp.dot(pool, wc1_ref[...], preferred_element_type=jnp.float32, precision=jax.lax.Precision.HIGHEST) + bc1_ref[...], 0.0)
    logits_ref[...] = jnp.dot(h1, wc2_ref[...], preferred_element_type=jnp.float32, precision=jax.lax.Precision.HIGHEST) + bc2_ref[...]
    e1 = jnp.maximum(
        jnp.dot(pool, we1_ref[...], preferred_element_type=jnp.float32, precision=jax.lax.Precision.HIGHEST) + be1_ref[...], 0.0)
    z = jnp.dot(e1, we2_ref[...], preferred_element_type=jnp.float32, precision=jax.lax.Precision.HIGHEST) + be2_ref[...]
    energy_ref[...] = jnp.maximum(z, 0.0) + jnp.log(1.0 + jnp.exp(-jnp.abs(z)))


def _heads(xsum, cnt, xmax, p):
    wpa = p["pool"]["W"][:, :HIDDEN].T
    wpb = p["pool"]["W"][:, HIDDEN:].T
    bp = p["pool"]["b"].reshape(1, -1)
    wc1 = p["cls1"]["W"].T
    bc1 = p["cls1"]["b"].reshape(1, -1)
    wc2 = p["cls2"]["W"].T
    bc2 = p["cls2"]["b"].reshape(1, -1)
    we1 = p["en1"]["W"].T
    be1 = p["en1"]["b"].reshape(1, -1)
    we2 = p["en2"]["W"].T
    be2 = p["en2"]["b"].reshape(1, -1)
    return pl.pallas_call(
        _head_body,
        out_shape=[
            jax.ShapeDtypeStruct((NUM_GRAPHS, 5), jnp.float32),
            jax.ShapeDtypeStruct((NUM_GRAPHS, 1), jnp.float32),
        ],
    )(xsum, cnt, xmax, wpa, wpb, bp, wc1, bc1, wc2, bc2, we1, be1, we2, be2)


def kernel(x, edge_index, batch, params):
    row = edge_index[0]
    col = edge_index[1]

    encw = params["enc_lin"]["W"].T
    s0, t0 = _bn_fold(params["enc_bn"])
    t0 = t0 + params["enc_lin"]["b"].reshape(1, -1) * s0
    h = _encoder(x, encw, s0, t0)

    for i in range(NUM_BLOCKS):
        pc = params["edge_convs"][i]
        w1 = pc["lin1"]["W"]
        w1aT = w1[:, :HIDDEN].T
        w1bT = w1[:, HIDDEN:].T
        s1, t1 = _bn_fold(pc["bn1"])
        t1 = t1 + pc["lin1"]["b"].reshape(1, -1) * s1
        w2T = pc["lin2"]["W"].T
        s2, t2 = _bn_fold(pc["bn2"])
        t2 = t2 + pc["lin2"]["b"].reshape(1, -1) * s2

        ec_parts = []
        for es, el in ERANGES:
            xr, xc = _sc_gather_multi([(h, row), (h, col)], es, el)
            he = _edge_mlp(xr, xc, w1aT, w1bT, s1, t1, w2T, s2, t2)
            ec_parts.append(jax.ops.segment_sum(he, row[es:es + el], num_segments=N))
        ln = params["lns"][2 * i]
        x1 = _res_ln(ec_parts, h, ln["g"].reshape(1, -1), ln["b"].reshape(1, -1))

        g = params["gats"][i]
        idx_h = jnp.repeat(jnp.arange(HEADS), HEAD_DIM)
        ms = jnp.zeros((HEADS, HIDDEN), jnp.float32)
        ms = ms.at[idx_h, jnp.arange(HIDDEN)].set(g["att_src"].reshape(-1))
        md = jnp.zeros((HEADS, HIDDEN), jnp.float32)
        md = md.at[idx_h, jnp.arange(HIDDEN)].set(g["att_dst"].reshape(-1))
        mT = jnp.concatenate([ms, md], axis=0).T
        haa, aa, cm = _gat_proj(x1, g["W"].T, mT)
        c = cm[:, :HEADS] + cm[:, HEADS:]
        rexp = jnp.zeros((HEADS, HIDDEN), jnp.float32)
        rexp = rexp.at[idx_h, jnp.arange(HIDDEN)].set(1.0)
        s_parts = []
        for es, el in ERANGES:
            har, ac = _sc_gather_multi([(haa, row), (aa, col)], es, el)
            y = _attn_msg(har, ac, c, rexp)
            s_parts.append(jax.ops.segment_sum(y, col[es:es + el], num_segments=N))
        ln2 = params["lns"][2 * i + 1]
        h = _gat_epilogue(s_parts, rexp, x1, g["bias"].reshape(1, -1),
                          g["ln_g"].reshape(1, -1), g["ln_b"].reshape(1, -1),
                          ln2["g"].reshape(1, -1), ln2["b"].reshape(1, -1))

    xsum, cnt, xmax = _pool(h, batch.reshape(-1, 1).astype(jnp.int32))
    logits, energy = _heads(xsum, cnt, xmax, params)
    return logits, energy
